# Initial kernel scaffold; baseline (speedup 1.0000x reference)
import jax
import jax.numpy as jnp
from jax.experimental import pallas as pl


def kernel(x, edge_index, edge_label_index, W0, b0, W1, b1, W2, b2, g0, be0, g1, be1, lpW1, lpb1, lpW2, lpb2, lpW3, lpb3):
    raise NotImplementedError("write your pallas kernel here")


# speedup vs baseline: 15.4125x; 15.4125x over previous
import functools

import jax
import jax.numpy as jnp
from jax import lax
from jax.experimental import pallas as pl
from jax.experimental.pallas import tpu as pltpu
from jax.experimental.pallas import tpu_sc as plsc

N = 10000
E = 320000
D = 128
H = 128
L = 65536
BN_EPS = 1e-5

NP = 10240
NC = 2
NS = 16
NW = NC * NS
SLAB = NP // NS

EPW = E // NW
CK = 100
NCH = EPW // CK

LPW = L // NW
CKL = 128
NCHL = LPW // CKL

_SC_MESH = plsc.VectorSubcoreMesh(
    core_axis_name="c", subcore_axis_name="s", num_cores=NC, num_subcores=NS)


@functools.partial(
    pl.kernel,
    mesh=_SC_MESH,
    out_type=jax.ShapeDtypeStruct((NC, NP), jnp.float32),
    scratch_types=[
        pltpu.VMEM((NCH, CK), jnp.int32),
        pltpu.VMEM((CK,), jnp.float32),
        pltpu.VMEM_SHARED((NP,), jnp.float32),
    ],
)
def _deg_kernel(dst_hbm, zero_hbm, ones_hbm, out_hbm, dst_v, ones_v, acc):
    c = lax.axis_index("c")
    s = lax.axis_index("s")
    tid = c * NS + s
    pltpu.sync_copy(dst_hbm.at[tid], dst_v)
    pltpu.sync_copy(ones_hbm, ones_v)
    slab = pl.ds(s * SLAB, SLAB)
    pltpu.sync_copy(zero_hbm.at[slab], acc.at[slab])
    plsc.subcore_barrier()

    def body(j, carry):
        pltpu.sync_copy(ones_v, acc.at[dst_v.at[j]], add=True)
        return carry

    lax.fori_loop(0, NCH, body, 0)
    plsc.subcore_barrier()
    pltpu.sync_copy(acc.at[slab], out_hbm.at[c, slab])


@functools.partial(
    pl.kernel,
    mesh=_SC_MESH,
    out_type=jax.ShapeDtypeStruct((NC, NP, D), jnp.float32),
    scratch_types=[
        pltpu.VMEM((NCH, CK), jnp.int32),
        pltpu.VMEM((NCH, CK), jnp.int32),
        pltpu.VMEM((CK, D), jnp.float32),
        pltpu.VMEM_SHARED((NP, D), jnp.float32),
        pltpu.SemaphoreType.DMA,
    ],
)
def _agg_kernel(p_hbm, src_hbm, dst_hbm, zero_hbm, out_hbm,
                src_v, dst_v, buf, acc, sem):
    c = lax.axis_index("c")
    s = lax.axis_index("s")
    tid = c * NS + s
    pltpu.sync_copy(src_hbm.at[tid], src_v)
    pltpu.sync_copy(dst_hbm.at[tid], dst_v)
    slab = pl.ds(s * SLAB, SLAB)

    @pl.when(c == 0)
    def _():
        pltpu.sync_copy(p_hbm.at[slab], acc.at[slab])

    @pl.when(c == 1)
    def _():
        pltpu.sync_copy(zero_hbm.at[slab], acc.at[slab])

    plsc.subcore_barrier()

    def body(j, carry):
        pltpu.async_copy(p_hbm.at[src_v.at[j]], buf, sem).wait()
        pltpu.sync_copy(buf, acc.at[dst_v.at[j]], add=True)
        return carry

    lax.fori_loop(0, NCH, body, 0)
    plsc.subcore_barrier()
    pltpu.sync_copy(acc.at[slab], out_hbm.at[c, slab])


@functools.partial(
    pl.kernel,
    mesh=_SC_MESH,
    out_type=(jax.ShapeDtypeStruct((L, D), jnp.float32),
              jax.ShapeDtypeStruct((L, D), jnp.float32)),
    scratch_types=[
        pltpu.VMEM((NCHL, CKL), jnp.int32),
        pltpu.VMEM((NCHL, CKL), jnp.int32),
        pltpu.VMEM((CKL, D), jnp.float32),
        pltpu.VMEM((CKL, D), jnp.float32),
        pltpu.SemaphoreType.DMA,
        pltpu.SemaphoreType.DMA,
    ],
)
def _decode_gather_kernel(zs_hbm, zd_hbm, src_hbm, dst_hbm, es_hbm, ed_hbm,
                          src_v, dst_v, bufa, bufb, sema, semb):
    c = lax.axis_index("c")
    s = lax.axis_index("s")
    tid = c * NS + s
    base = tid * LPW
    pltpu.sync_copy(src_hbm.at[tid], src_v)
    pltpu.sync_copy(dst_hbm.at[tid], dst_v)

    def body(j, carry):
        rows = pl.ds(base + j * CKL, CKL)
        ga = pltpu.async_copy(zs_hbm.at[src_v.at[j]], bufa, sema)
        gb = pltpu.async_copy(zd_hbm.at[dst_v.at[j]], bufb, semb)
        ga.wait()
        pltpu.sync_copy(bufa, es_hbm.at[rows])
        gb.wait()
        pltpu.sync_copy(bufb, ed_hbm.at[rows])
        return carry

    lax.fori_loop(0, NCHL, body, 0)


_R = 1024


def _tc1_body(x_ref, w_ref, degt_ref, p_ref, dinv_ref):
    deg = degt_ref[:, 0:1] + degt_ref[:, 1:2] + 1.0
    di = lax.rsqrt(deg)
    h = jnp.dot(x_ref[...], w_ref[...], preferred_element_type=jnp.float32)
    dinv_ref[...] = di
    p_ref[...] = di * h


_tc1 = pl.pallas_call(
    _tc1_body,
    grid=(NP // _R,),
    in_specs=[
        pl.BlockSpec((_R, D), lambda i: (i, 0)),
        pl.BlockSpec((D, H), lambda i: (0, 0)),
        pl.BlockSpec((_R, 2), lambda i: (i, 0)),
    ],
    out_specs=[
        pl.BlockSpec((_R, H), lambda i: (i, 0)),
        pl.BlockSpec((_R, 1), lambda i: (i, 0)),
    ],
    out_shape=[
        jax.ShapeDtypeStruct((NP, H), jnp.float32),
        jax.ShapeDtypeStruct((NP, 1), jnp.float32),
    ],
)


def _tc_layer_body(a0_ref, a1_ref, dinv_ref, b_ref, g_ref, be_ref, w_ref,
                   p_ref):
    di = dinv_ref[...]
    conv = di * (a0_ref[...] + a1_ref[...]) + b_ref[...]
    bn_scale = g_ref[...] * (1.0 / (1.0 + BN_EPS) ** 0.5)
    r = jnp.maximum(conv * bn_scale + be_ref[...], 0.0)
    p_ref[...] = di * jnp.dot(r, w_ref[...],
                              preferred_element_type=jnp.float32)


_tc_layer = pl.pallas_call(
    _tc_layer_body,
    grid=(NP // _R,),
    in_specs=[
        pl.BlockSpec((_R, D), lambda i: (i, 0)),
        pl.BlockSpec((_R, D), lambda i: (i, 0)),
        pl.BlockSpec((_R, 1), lambda i: (i, 0)),
        pl.BlockSpec((1, H), lambda i: (0, 0)),
        pl.BlockSpec((1, H), lambda i: (0, 0)),
        pl.BlockSpec((1, H), lambda i: (0, 0)),
        pl.BlockSpec((H, H), lambda i: (0, 0)),
    ],
    out_specs=pl.BlockSpec((_R, H), lambda i: (i, 0)),
    out_shape=jax.ShapeDtypeStruct((NP, H), jnp.float32),
)


def _tc_z_body(a0_ref, a1_ref, dinv_ref, b_ref, wa_ref, wb_ref, lpb1_ref,
               zs_ref, zd_ref):
    z = dinv_ref[...] * (a0_ref[...] + a1_ref[...]) + b_ref[...]
    zs_ref[...] = jnp.dot(z, wa_ref[...],
                          preferred_element_type=jnp.float32) + lpb1_ref[...]
    zd_ref[...] = jnp.dot(z, wb_ref[...], preferred_element_type=jnp.float32)


_tc_z = pl.pallas_call(
    _tc_z_body,
    grid=(NP // _R,),
    in_specs=[
        pl.BlockSpec((_R, D), lambda i: (i, 0)),
        pl.BlockSpec((_R, D), lambda i: (i, 0)),
        pl.BlockSpec((_R, 1), lambda i: (i, 0)),
        pl.BlockSpec((1, H), lambda i: (0, 0)),
        pl.BlockSpec((H, H), lambda i: (0, 0)),
        pl.BlockSpec((H, H), lambda i: (0, 0)),
        pl.BlockSpec((1, H), lambda i: (0, 0)),
    ],
    out_specs=[
        pl.BlockSpec((_R, H), lambda i: (i, 0)),
        pl.BlockSpec((_R, H), lambda i: (i, 0)),
    ],
    out_shape=[
        jax.ShapeDtypeStruct((NP, H), jnp.float32),
        jax.ShapeDtypeStruct((NP, H), jnp.float32),
    ],
)

_RL = 2048


def _dec_mlp_body(es_ref, ed_ref, w2_ref, b2_ref, w3_ref, b3_ref, out_ref):
    t = jnp.maximum(es_ref[...] + ed_ref[...], 0.0)
    t2 = jnp.maximum(
        jnp.dot(t, w2_ref[...], preferred_element_type=jnp.float32)
        + b2_ref[...], 0.0)
    out_ref[...] = jnp.dot(t2, w3_ref[...],
                           preferred_element_type=jnp.float32) + b3_ref[...]


_dec_mlp = pl.pallas_call(
    _dec_mlp_body,
    grid=(L // _RL,),
    in_specs=[
        pl.BlockSpec((_RL, D), lambda i: (i, 0)),
        pl.BlockSpec((_RL, D), lambda i: (i, 0)),
        pl.BlockSpec((H, H // 2), lambda i: (0, 0)),
        pl.BlockSpec((1, H // 2), lambda i: (0, 0)),
        pl.BlockSpec((H // 2, 1), lambda i: (0, 0)),
        pl.BlockSpec((1, 1), lambda i: (0, 0)),
    ],
    out_specs=pl.BlockSpec((_RL, 1), lambda i: (i, 0)),
    out_shape=jax.ShapeDtypeStruct((L, 1), jnp.float32),
)


def kernel(x, edge_index, edge_label_index, W0, b0, W1, b1, W2, b2,
           g0, be0, g1, be1, lpW1, lpb1, lpW2, lpb2, lpW3, lpb3):
    x_p = jnp.zeros((NP, D), jnp.float32).at[:N].set(x)
    src3 = edge_index[0].reshape(NW, NCH, CK)
    dst3 = edge_index[1].reshape(NW, NCH, CK)
    srcl3 = edge_label_index[0].reshape(NW, NCHL, CKL)
    dstl3 = edge_label_index[1].reshape(NW, NCHL, CKL)

    zero_np = jnp.zeros((NP,), jnp.float32)
    zero_nd = jnp.zeros((NP, D), jnp.float32)
    ones_ck = jnp.ones((CK,), jnp.float32)

    degp = _deg_kernel(dst3, zero_np, ones_ck)
    degt = degp.T

    p0, dinv = _tc1(x_p, W0, degt)
    a = _agg_kernel(p0, src3, dst3, zero_nd)
    p1 = _tc_layer(a[0], a[1], dinv, b0.reshape(1, H), g0.reshape(1, H),
                   be0.reshape(1, H), W1)
    a = _agg_kernel(p1, src3, dst3, zero_nd)
    p2 = _tc_layer(a[0], a[1], dinv, b1.reshape(1, H), g1.reshape(1, H),
                   be1.reshape(1, H), W2)
    a = _agg_kernel(p2, src3, dst3, zero_nd)
    zs, zd = _tc_z(a[0], a[1], dinv, b2.reshape(1, H), lpW1[:H], lpW1[H:],
                   lpb1.reshape(1, H))

    es, ed = _decode_gather_kernel(zs, zd, srcl3, dstl3)
    out = _dec_mlp(es, ed, lpW2, lpb2.reshape(1, H // 2), lpW3,
                   lpb3.reshape(1, 1))
    return out[:, 0]


# speedup vs baseline: 22.0538x; 1.4309x over previous
import functools

import jax
import jax.numpy as jnp
from jax import lax
from jax.experimental import pallas as pl
from jax.experimental.pallas import tpu as pltpu
from jax.experimental.pallas import tpu_sc as plsc

N = 10000
E = 320000
D = 128
H = 128
HH = H // 2
L = 65536
BN_EPS = 1e-5

NP = 10240
NC = 2
NS = 16
NW = NC * NS
SLAB = NP // NS

EPW = E // NW
CK = 125
NCH = EPW // CK
NBC = 16
NB = NCH // NBC
CKD = 100
NCHD = EPW // CKD

LPW = L // NW
CKL = 128
NCHL = LPW // CKL

_SC_MESH = plsc.VectorSubcoreMesh(
    core_axis_name="c", subcore_axis_name="s", num_cores=NC, num_subcores=NS)


@functools.partial(
    pl.kernel,
    mesh=_SC_MESH,
    out_type=jax.ShapeDtypeStruct((NC, NP), jnp.float32),
    scratch_types=[
        pltpu.VMEM((NCHD, CKD), jnp.int32),
        pltpu.VMEM((CKD,), jnp.float32),
        pltpu.VMEM_SHARED((NP,), jnp.float32),
    ],
)
def _deg_kernel(dst_hbm, zero_hbm, ones_hbm, out_hbm, dst_v, ones_v, acc):
    c = lax.axis_index("c")
    s = lax.axis_index("s")
    tid = c * NS + s
    pltpu.sync_copy(dst_hbm.at[tid], dst_v)
    pltpu.sync_copy(ones_hbm, ones_v)
    slab = pl.ds(s * SLAB, SLAB)
    pltpu.sync_copy(zero_hbm.at[slab], acc.at[slab])
    plsc.subcore_barrier()

    def body(j, carry):
        pltpu.sync_copy(ones_v, acc.at[dst_v.at[j]], add=True)
        return carry

    lax.fori_loop(0, NCHD, body, 0)
    plsc.subcore_barrier()
    pltpu.sync_copy(acc.at[slab], out_hbm.at[c, slab])


@functools.partial(
    pl.kernel,
    mesh=_SC_MESH,
    out_type=jax.ShapeDtypeStruct((NC, NP, D), jnp.float32),
    scratch_types=[
        pltpu.VMEM((NBC, CK), jnp.int32),
        pltpu.VMEM((NBC, CK), jnp.int32),
        pltpu.VMEM((CK, D), jnp.float32),
        pltpu.VMEM((CK, D), jnp.float32),
        pltpu.VMEM_SHARED((NP, D), jnp.float32),
        pltpu.SemaphoreType.DMA,
        pltpu.SemaphoreType.DMA,
    ],
)
def _agg_kernel(p_hbm, src_hbm, dst_hbm, zero_hbm, out_hbm,
                src_v, dst_v, buf0, buf1, acc, sem0, sem1):
    c = lax.axis_index("c")
    s = lax.axis_index("s")
    tid = c * NS + s
    slab = pl.ds(s * SLAB, SLAB)

    @pl.when(c == 0)
    def _():
        pltpu.sync_copy(p_hbm.at[slab], acc.at[slab])

    @pl.when(c == 1)
    def _():
        pltpu.sync_copy(zero_hbm.at[slab], acc.at[slab])

    plsc.subcore_barrier()

    def block(b, carry):
        rows = pl.ds(b * NBC, NBC)
        pltpu.sync_copy(src_hbm.at[tid, rows], src_v)
        pltpu.sync_copy(dst_hbm.at[tid, rows], dst_v)
        pltpu.async_copy(p_hbm.at[src_v.at[0]], buf0, sem0)

        def body(k, carry2):
            j0 = 2 * k
            g1 = pltpu.async_copy(p_hbm.at[src_v.at[j0 + 1]], buf1, sem1)
            pltpu.make_async_copy(p_hbm.at[src_v.at[j0]], buf0, sem0).wait()
            pltpu.sync_copy(buf0, acc.at[dst_v.at[j0]], add=True)
            pltpu.async_copy(p_hbm.at[src_v.at[j0 + 2]], buf0, sem0)
            g1.wait()
            pltpu.sync_copy(buf1, acc.at[dst_v.at[j0 + 1]], add=True)
            return carry2

        lax.fori_loop(0, NBC // 2 - 1, body, 0)
        gl = pltpu.async_copy(p_hbm.at[src_v.at[NBC - 1]], buf1, sem1)
        pltpu.make_async_copy(p_hbm.at[src_v.at[NBC - 2]], buf0, sem0).wait()
        pltpu.sync_copy(buf0, acc.at[dst_v.at[NBC - 2]], add=True)
        gl.wait()
        pltpu.sync_copy(buf1, acc.at[dst_v.at[NBC - 1]], add=True)
        return carry

    lax.fori_loop(0, NB, block, 0)
    plsc.subcore_barrier()
    pltpu.sync_copy(acc.at[slab], out_hbm.at[c, slab])


@functools.partial(
    pl.kernel,
    mesh=_SC_MESH,
    out_type=(jax.ShapeDtypeStruct((L, D), jnp.float32),
              jax.ShapeDtypeStruct((L, D), jnp.float32)),
    scratch_types=[
        pltpu.VMEM((NCHL, CKL), jnp.int32),
        pltpu.VMEM((NCHL, CKL), jnp.int32),
        pltpu.VMEM((CKL, D), jnp.float32),
        pltpu.VMEM((CKL, D), jnp.float32),
        pltpu.VMEM((CKL, D), jnp.float32),
        pltpu.VMEM((CKL, D), jnp.float32),
        pltpu.SemaphoreType.DMA,
        pltpu.SemaphoreType.DMA,
        pltpu.SemaphoreType.DMA,
        pltpu.SemaphoreType.DMA,
    ],
)
def _decode_gather_kernel(zs_hbm, zd_hbm, src_hbm, dst_hbm, es_hbm, ed_hbm,
                          src_v, dst_v, bufa0, bufb0, bufa1, bufb1,
                          sa0, sb0, sa1, sb1):
    c = lax.axis_index("c")
    s = lax.axis_index("s")
    tid = c * NS + s
    base = tid * LPW
    pltpu.sync_copy(src_hbm.at[tid], src_v)
    pltpu.sync_copy(dst_hbm.at[tid], dst_v)

    pltpu.async_copy(zs_hbm.at[src_v.at[0]], bufa0, sa0)
    pltpu.async_copy(zd_hbm.at[dst_v.at[0]], bufb0, sb0)

    def emit(j, bufa, bufb, sa, sb):
        rows = pl.ds(base + j * CKL, CKL)
        pltpu.make_async_copy(zs_hbm.at[src_v.at[j]], bufa, sa).wait()
        pltpu.sync_copy(bufa, es_hbm.at[rows])
        pltpu.make_async_copy(zd_hbm.at[dst_v.at[j]], bufb, sb).wait()
        pltpu.sync_copy(bufb, ed_hbm.at[rows])

    def body(k, carry):
        j0 = 2 * k
        pltpu.async_copy(zs_hbm.at[src_v.at[j0 + 1]], bufa1, sa1)
        pltpu.async_copy(zd_hbm.at[dst_v.at[j0 + 1]], bufb1, sb1)
        emit(j0, bufa0, bufb0, sa0, sb0)
        pltpu.async_copy(zs_hbm.at[src_v.at[j0 + 2]], bufa0, sa0)
        pltpu.async_copy(zd_hbm.at[dst_v.at[j0 + 2]], bufb0, sb0)
        emit(j0 + 1, bufa1, bufb1, sa1, sb1)
        return carry

    lax.fori_loop(0, NCHL // 2 - 1, body, 0)
    pltpu.async_copy(zs_hbm.at[src_v.at[NCHL - 1]], bufa1, sa1)
    pltpu.async_copy(zd_hbm.at[dst_v.at[NCHL - 1]], bufb1, sb1)
    emit(NCHL - 2, bufa0, bufb0, sa0, sb0)
    emit(NCHL - 1, bufa1, bufb1, sa1, sb1)


_R = 1024


def _tc1_body(x_ref, w_ref, degt_ref, p_ref, dinv_ref):
    deg = degt_ref[:, 0:1] + degt_ref[:, 1:2] + 1.0
    di = lax.rsqrt(deg)
    h = jnp.dot(x_ref[...], w_ref[...], preferred_element_type=jnp.float32)
    dinv_ref[...] = di
    p_ref[...] = di * h


_tc1 = pl.pallas_call(
    _tc1_body,
    grid=(NP // _R,),
    in_specs=[
        pl.BlockSpec((_R, D), lambda i: (i, 0)),
        pl.BlockSpec((D, H), lambda i: (0, 0)),
        pl.BlockSpec((_R, 2), lambda i: (i, 0)),
    ],
    out_specs=[
        pl.BlockSpec((_R, H), lambda i: (i, 0)),
        pl.BlockSpec((_R, 1), lambda i: (i, 0)),
    ],
    out_shape=[
        jax.ShapeDtypeStruct((NP, H), jnp.float32),
        jax.ShapeDtypeStruct((NP, 1), jnp.float32),
    ],
)


def _tc_layer_body(a_ref, dinv_ref, b_ref, g_ref, be_ref, w_ref, p_ref):
    di = dinv_ref[...]
    blk = a_ref[...]
    conv = di * (blk[0] + blk[1]) + b_ref[...]
    bn_scale = g_ref[...] * (1.0 / (1.0 + BN_EPS) ** 0.5)
    r = jnp.maximum(conv * bn_scale + be_ref[...], 0.0)
    p_ref[...] = di * jnp.dot(r, w_ref[...],
                              preferred_element_type=jnp.float32)


_tc_layer = pl.pallas_call(
    _tc_layer_body,
    grid=(NP // _R,),
    in_specs=[
        pl.BlockSpec((NC, _R, D), lambda i: (0, i, 0)),
        pl.BlockSpec((_R, 1), lambda i: (i, 0)),
        pl.BlockSpec((1, H), lambda i: (0, 0)),
        pl.BlockSpec((1, H), lambda i: (0, 0)),
        pl.BlockSpec((1, H), lambda i: (0, 0)),
        pl.BlockSpec((H, H), lambda i: (0, 0)),
    ],
    out_specs=pl.BlockSpec((_R, H), lambda i: (i, 0)),
    out_shape=jax.ShapeDtypeStruct((NP, H), jnp.float32),
)


def _tc_z_body(a_ref, dinv_ref, b_ref, wa_ref, wb_ref, lpb1_ref,
               zs_ref, zd_ref):
    blk = a_ref[...]
    z = dinv_ref[...] * (blk[0] + blk[1]) + b_ref[...]
    zs_ref[...] = jnp.dot(z, wa_ref[...],
                          preferred_element_type=jnp.float32) + lpb1_ref[...]
    zd_ref[...] = jnp.dot(z, wb_ref[...], preferred_element_type=jnp.float32)


_tc_z = pl.pallas_call(
    _tc_z_body,
    grid=(NP // _R,),
    in_specs=[
        pl.BlockSpec((NC, _R, D), lambda i: (0, i, 0)),
        pl.BlockSpec((_R, 1), lambda i: (i, 0)),
        pl.BlockSpec((1, H), lambda i: (0, 0)),
        pl.BlockSpec((H, H), lambda i: (0, 0)),
        pl.BlockSpec((H, H), lambda i: (0, 0)),
        pl.BlockSpec((1, H), lambda i: (0, 0)),
    ],
    out_specs=[
        pl.BlockSpec((_R, H), lambda i: (i, 0)),
        pl.BlockSpec((_R, H), lambda i: (i, 0)),
    ],
    out_shape=[
        jax.ShapeDtypeStruct((NP, H), jnp.float32),
        jax.ShapeDtypeStruct((NP, H), jnp.float32),
    ],
)

_RL = 2048


def _dec_mlp_body(es_ref, ed_ref, w2_ref, b2_ref, w3_ref, b3_ref, out_ref):
    t = jnp.maximum(es_ref[...] + ed_ref[...], 0.0)
    t2 = jnp.maximum(
        jnp.dot(t, w2_ref[...], preferred_element_type=jnp.float32)
        + b2_ref[...], 0.0)
    out_ref[...] = jnp.dot(t2, w3_ref[...],
                           preferred_element_type=jnp.float32) + b3_ref[...]


_dec_mlp = pl.pallas_call(
    _dec_mlp_body,
    grid=(L // _RL,),
    in_specs=[
        pl.BlockSpec((_RL, D), lambda i: (i, 0)),
        pl.BlockSpec((_RL, D), lambda i: (i, 0)),
        pl.BlockSpec((H, HH), lambda i: (0, 0)),
        pl.BlockSpec((1, HH), lambda i: (0, 0)),
        pl.BlockSpec((HH, 1), lambda i: (0, 0)),
        pl.BlockSpec((1, 1), lambda i: (0, 0)),
    ],
    out_specs=pl.BlockSpec((_RL, 1), lambda i: (i, 0)),
    out_shape=jax.ShapeDtypeStruct((L, 1), jnp.float32),
)


def kernel(x, edge_index, edge_label_index, W0, b0, W1, b1, W2, b2,
           g0, be0, g1, be1, lpW1, lpb1, lpW2, lpb2, lpW3, lpb3):
    x_p = jnp.zeros((NP, D), jnp.float32).at[:N].set(x)
    src3 = edge_index[0].reshape(NW, NCH, CK)
    dst3 = edge_index[1].reshape(NW, NCH, CK)
    dst3d = edge_index[1].reshape(NW, NCHD, CKD)
    srcl3 = edge_label_index[0].reshape(NW, NCHL, CKL)
    dstl3 = edge_label_index[1].reshape(NW, NCHL, CKL)

    zero_np = jnp.zeros((NP,), jnp.float32)
    zero_nd = jnp.zeros((NP, D), jnp.float32)
    ones_ck = jnp.ones((CKD,), jnp.float32)

    degp = _deg_kernel(dst3d, zero_np, ones_ck)
    degt = degp.T

    p0, dinv = _tc1(x_p, W0, degt)
    a = _agg_kernel(p0, src3, dst3, zero_nd)
    p1 = _tc_layer(a, dinv, b0.reshape(1, H), g0.reshape(1, H),
                   be0.reshape(1, H), W1)
    a = _agg_kernel(p1, src3, dst3, zero_nd)
    p2 = _tc_layer(a, dinv, b1.reshape(1, H), g1.reshape(1, H),
                   be1.reshape(1, H), W2)
    a = _agg_kernel(p2, src3, dst3, zero_nd)
    zs, zd = _tc_z(a, dinv, b2.reshape(1, H), lpW1[:H], lpW1[H:],
                   lpb1.reshape(1, H))

    es, ed = _decode_gather_kernel(zs, zd, srcl3, dstl3)
    out = _dec_mlp(es, ed, lpW2, lpb2.reshape(1, HH), lpW3,
                   lpb3.reshape(1, 1))
    return out[:, 0]


# speedup vs baseline: 22.8238x; 1.0349x over previous
import functools

import jax
import jax.numpy as jnp
from jax import lax
from jax.experimental import pallas as pl
from jax.experimental.pallas import tpu as pltpu
from jax.experimental.pallas import tpu_sc as plsc

N = 10000
E = 320000
D = 128
H = 128
HH = H // 2
L = 65536
BN_EPS = 1e-5

NP = 10240
NC = 2
NS = 16
NW = NC * NS
SLAB = NP // NS

EPW = E // NW
CK = 125
NCH = EPW // CK
NBC = 40
NB = NCH // NBC
CKD = 100
NCHD = EPW // CKD

LPW = L // NW
CKL = 128
NCHL = LPW // CKL

_SC_MESH = plsc.VectorSubcoreMesh(
    core_axis_name="c", subcore_axis_name="s", num_cores=NC, num_subcores=NS)


@functools.partial(
    pl.kernel,
    mesh=_SC_MESH,
    out_type=jax.ShapeDtypeStruct((NC, NP), jnp.float32),
    scratch_types=[
        pltpu.VMEM((NCHD, CKD), jnp.int32),
        pltpu.VMEM((CKD,), jnp.float32),
        pltpu.VMEM_SHARED((NP,), jnp.float32),
    ],
)
def _deg_kernel(dst_hbm, zero_hbm, ones_hbm, out_hbm, dst_v, ones_v, acc):
    c = lax.axis_index("c")
    s = lax.axis_index("s")
    tid = c * NS + s
    pltpu.sync_copy(dst_hbm.at[tid], dst_v)
    pltpu.sync_copy(ones_hbm, ones_v)
    slab = pl.ds(s * SLAB, SLAB)
    pltpu.sync_copy(zero_hbm.at[slab], acc.at[slab])
    plsc.subcore_barrier()

    def body(j, carry):
        pltpu.sync_copy(ones_v, acc.at[dst_v.at[j]], add=True)
        return carry

    lax.fori_loop(0, NCHD, body, 0)
    plsc.subcore_barrier()
    pltpu.sync_copy(acc.at[slab], out_hbm.at[c, slab])


@functools.partial(
    pl.kernel,
    mesh=_SC_MESH,
    out_type=jax.ShapeDtypeStruct((NC, NP, D), jnp.float32),
    scratch_types=[
        pltpu.VMEM((NBC, CK), jnp.int32),
        pltpu.VMEM((NBC, CK), jnp.int32),
        pltpu.VMEM((CK, D), jnp.float32),
        pltpu.VMEM((CK, D), jnp.float32),
        pltpu.VMEM_SHARED((NP, D), jnp.float32),
        pltpu.SemaphoreType.DMA,
        pltpu.SemaphoreType.DMA,
    ],
)
def _agg_kernel(p_hbm, src_hbm, dst_hbm, zero_hbm, out_hbm,
                src_v, dst_v, buf0, buf1, acc, sem0, sem1):
    c = lax.axis_index("c")
    s = lax.axis_index("s")
    tid = c * NS + s
    slab = pl.ds(s * SLAB, SLAB)

    @pl.when(c == 0)
    def _():
        pltpu.sync_copy(p_hbm.at[slab], acc.at[slab])

    @pl.when(c == 1)
    def _():
        pltpu.sync_copy(zero_hbm.at[slab], acc.at[slab])

    plsc.subcore_barrier()

    def block(b, carry):
        rows = pl.ds(b * NBC, NBC)
        pltpu.sync_copy(src_hbm.at[tid, rows], src_v)
        pltpu.sync_copy(dst_hbm.at[tid, rows], dst_v)
        pltpu.async_copy(p_hbm.at[src_v.at[0]], buf0, sem0)

        def body(k, carry2):
            j0 = 2 * k
            g1 = pltpu.async_copy(p_hbm.at[src_v.at[j0 + 1]], buf1, sem1)
            pltpu.make_async_copy(p_hbm.at[src_v.at[j0]], buf0, sem0).wait()
            pltpu.sync_copy(buf0, acc.at[dst_v.at[j0]], add=True)
            pltpu.async_copy(p_hbm.at[src_v.at[j0 + 2]], buf0, sem0)
            g1.wait()
            pltpu.sync_copy(buf1, acc.at[dst_v.at[j0 + 1]], add=True)
            return carry2

        lax.fori_loop(0, NBC // 2 - 1, body, 0)
        gl = pltpu.async_copy(p_hbm.at[src_v.at[NBC - 1]], buf1, sem1)
        pltpu.make_async_copy(p_hbm.at[src_v.at[NBC - 2]], buf0, sem0).wait()
        pltpu.sync_copy(buf0, acc.at[dst_v.at[NBC - 2]], add=True)
        gl.wait()
        pltpu.sync_copy(buf1, acc.at[dst_v.at[NBC - 1]], add=True)
        return carry

    lax.fori_loop(0, NB, block, 0)
    plsc.subcore_barrier()
    pltpu.sync_copy(acc.at[slab], out_hbm.at[c, slab])


@functools.partial(
    pl.kernel,
    mesh=_SC_MESH,
    out_type=(jax.ShapeDtypeStruct((L, D), jnp.float32),
              jax.ShapeDtypeStruct((L, D), jnp.float32)),
    scratch_types=[
        pltpu.VMEM((NCHL, CKL), jnp.int32),
        pltpu.VMEM((NCHL, CKL), jnp.int32),
        pltpu.VMEM((CKL, D), jnp.float32),
        pltpu.VMEM((CKL, D), jnp.float32),
        pltpu.VMEM((CKL, D), jnp.float32),
        pltpu.VMEM((CKL, D), jnp.float32),
        pltpu.SemaphoreType.DMA,
        pltpu.SemaphoreType.DMA,
        pltpu.SemaphoreType.DMA,
        pltpu.SemaphoreType.DMA,
    ],
)
def _decode_gather_kernel(zs_hbm, zd_hbm, src_hbm, dst_hbm, es_hbm, ed_hbm,
                          src_v, dst_v, bufa0, bufb0, bufa1, bufb1,
                          sa0, sb0, sa1, sb1):
    c = lax.axis_index("c")
    s = lax.axis_index("s")
    tid = c * NS + s
    base = tid * LPW
    pltpu.sync_copy(src_hbm.at[tid], src_v)
    pltpu.sync_copy(dst_hbm.at[tid], dst_v)

    pltpu.async_copy(zs_hbm.at[src_v.at[0]], bufa0, sa0)
    pltpu.async_copy(zd_hbm.at[dst_v.at[0]], bufb0, sb0)

    def emit(j, bufa, bufb, sa, sb):
        rows = pl.ds(base + j * CKL, CKL)
        pltpu.make_async_copy(zs_hbm.at[src_v.at[j]], bufa, sa).wait()
        pltpu.sync_copy(bufa, es_hbm.at[rows])
        pltpu.make_async_copy(zd_hbm.at[dst_v.at[j]], bufb, sb).wait()
        pltpu.sync_copy(bufb, ed_hbm.at[rows])

    def body(k, carry):
        j0 = 2 * k
        pltpu.async_copy(zs_hbm.at[src_v.at[j0 + 1]], bufa1, sa1)
        pltpu.async_copy(zd_hbm.at[dst_v.at[j0 + 1]], bufb1, sb1)
        emit(j0, bufa0, bufb0, sa0, sb0)
        pltpu.async_copy(zs_hbm.at[src_v.at[j0 + 2]], bufa0, sa0)
        pltpu.async_copy(zd_hbm.at[dst_v.at[j0 + 2]], bufb0, sb0)
        emit(j0 + 1, bufa1, bufb1, sa1, sb1)
        return carry

    lax.fori_loop(0, NCHL // 2 - 1, body, 0)
    pltpu.async_copy(zs_hbm.at[src_v.at[NCHL - 1]], bufa1, sa1)
    pltpu.async_copy(zd_hbm.at[dst_v.at[NCHL - 1]], bufb1, sb1)
    emit(NCHL - 2, bufa0, bufb0, sa0, sb0)
    emit(NCHL - 1, bufa1, bufb1, sa1, sb1)


_R = 1024


def _tc_mm0_body(x_ref, w_ref, h_ref):
    h_ref[...] = jnp.dot(x_ref[...], w_ref[...],
                         preferred_element_type=jnp.float32)


_tc_mm0 = pl.pallas_call(
    _tc_mm0_body,
    grid=(NP // _R,),
    in_specs=[
        pl.BlockSpec((_R, D), lambda i: (i, 0)),
        pl.BlockSpec((D, H), lambda i: (0, 0)),
    ],
    out_specs=pl.BlockSpec((_R, H), lambda i: (i, 0)),
    out_shape=jax.ShapeDtypeStruct((NP, H), jnp.float32),
)


def _tc_scale_body(h_ref, degt_ref, p_ref, dinv_ref):
    deg = degt_ref[:, 0:1] + degt_ref[:, 1:2] + 1.0
    di = lax.rsqrt(deg)
    dinv_ref[...] = di
    p_ref[...] = di * h_ref[...]


_tc_scale = pl.pallas_call(
    _tc_scale_body,
    grid=(NP // _R,),
    in_specs=[
        pl.BlockSpec((_R, H), lambda i: (i, 0)),
        pl.BlockSpec((_R, 2), lambda i: (i, 0)),
    ],
    out_specs=[
        pl.BlockSpec((_R, H), lambda i: (i, 0)),
        pl.BlockSpec((_R, 1), lambda i: (i, 0)),
    ],
    out_shape=[
        jax.ShapeDtypeStruct((NP, H), jnp.float32),
        jax.ShapeDtypeStruct((NP, 1), jnp.float32),
    ],
)


def _tc_layer_body(a_ref, dinv_ref, b_ref, g_ref, be_ref, w_ref, p_ref):
    di = dinv_ref[...]
    blk = a_ref[...]
    conv = di * (blk[0] + blk[1]) + b_ref[...]
    bn_scale = g_ref[...] * (1.0 / (1.0 + BN_EPS) ** 0.5)
    r = jnp.maximum(conv * bn_scale + be_ref[...], 0.0)
    p_ref[...] = di * jnp.dot(r, w_ref[...],
                              preferred_element_type=jnp.float32)


_tc_layer = pl.pallas_call(
    _tc_layer_body,
    grid=(NP // _R,),
    in_specs=[
        pl.BlockSpec((NC, _R, D), lambda i: (0, i, 0)),
        pl.BlockSpec((_R, 1), lambda i: (i, 0)),
        pl.BlockSpec((1, H), lambda i: (0, 0)),
        pl.BlockSpec((1, H), lambda i: (0, 0)),
        pl.BlockSpec((1, H), lambda i: (0, 0)),
        pl.BlockSpec((H, H), lambda i: (0, 0)),
    ],
    out_specs=pl.BlockSpec((_R, H), lambda i: (i, 0)),
    out_shape=jax.ShapeDtypeStruct((NP, H), jnp.float32),
)


def _tc_z_body(a_ref, dinv_ref, b_ref, wa_ref, wb_ref, lpb1_ref,
               zs_ref, zd_ref):
    blk = a_ref[...]
    z = dinv_ref[...] * (blk[0] + blk[1]) + b_ref[...]
    zs_ref[...] = jnp.dot(z, wa_ref[...],
                          preferred_element_type=jnp.float32) + lpb1_ref[...]
    zd_ref[...] = jnp.dot(z, wb_ref[...], preferred_element_type=jnp.float32)


_tc_z = pl.pallas_call(
    _tc_z_body,
    grid=(NP // _R,),
    in_specs=[
        pl.BlockSpec((NC, _R, D), lambda i: (0, i, 0)),
        pl.BlockSpec((_R, 1), lambda i: (i, 0)),
        pl.BlockSpec((1, H), lambda i: (0, 0)),
        pl.BlockSpec((H, H), lambda i: (0, 0)),
        pl.BlockSpec((H, H), lambda i: (0, 0)),
        pl.BlockSpec((1, H), lambda i: (0, 0)),
    ],
    out_specs=[
        pl.BlockSpec((_R, H), lambda i: (i, 0)),
        pl.BlockSpec((_R, H), lambda i: (i, 0)),
    ],
    out_shape=[
        jax.ShapeDtypeStruct((NP, H), jnp.float32),
        jax.ShapeDtypeStruct((NP, H), jnp.float32),
    ],
)

_RL = 2048


def _dec_mlp_body(es_ref, ed_ref, w2_ref, b2_ref, w3_ref, b3_ref, out_ref):
    t = jnp.maximum(es_ref[...] + ed_ref[...], 0.0)
    t2 = jnp.maximum(
        jnp.dot(t, w2_ref[...], preferred_element_type=jnp.float32)
        + b2_ref[...], 0.0)
    out_ref[...] = jnp.dot(t2, w3_ref[...],
                           preferred_element_type=jnp.float32) + b3_ref[...]


_dec_mlp = pl.pallas_call(
    _dec_mlp_body,
    grid=(L // _RL,),
    in_specs=[
        pl.BlockSpec((_RL, D), lambda i: (i, 0)),
        pl.BlockSpec((_RL, D), lambda i: (i, 0)),
        pl.BlockSpec((H, HH), lambda i: (0, 0)),
        pl.BlockSpec((1, HH), lambda i: (0, 0)),
        pl.BlockSpec((HH, 1), lambda i: (0, 0)),
        pl.BlockSpec((1, 1), lambda i: (0, 0)),
    ],
    out_specs=pl.BlockSpec((_RL, 1), lambda i: (i, 0)),
    out_shape=jax.ShapeDtypeStruct((L, 1), jnp.float32),
)


def kernel(x, edge_index, edge_label_index, W0, b0, W1, b1, W2, b2,
           g0, be0, g1, be1, lpW1, lpb1, lpW2, lpb2, lpW3, lpb3):
    x_p = jnp.zeros((NP, D), jnp.float32).at[:N].set(x)
    src3 = edge_index[0].reshape(NW, NCH, CK)
    dst3 = edge_index[1].reshape(NW, NCH, CK)
    dst3d = edge_index[1].reshape(NW, NCHD, CKD)
    srcl3 = edge_label_index[0].reshape(NW, NCHL, CKL)
    dstl3 = edge_label_index[1].reshape(NW, NCHL, CKL)

    zero_np = jnp.zeros((NP,), jnp.float32)
    zero_nd = jnp.zeros((NP, D), jnp.float32)
    ones_ck = jnp.ones((CKD,), jnp.float32)

    h0 = _tc_mm0(x_p, W0)
    degp = _deg_kernel(dst3d, zero_np, ones_ck)
    degt = degp.T

    p0, dinv = _tc_scale(h0, degt)
    a = _agg_kernel(p0, src3, dst3, zero_nd)
    p1 = _tc_layer(a, dinv, b0.reshape(1, H), g0.reshape(1, H),
                   be0.reshape(1, H), W1)
    a = _agg_kernel(p1, src3, dst3, zero_nd)
    p2 = _tc_layer(a, dinv, b1.reshape(1, H), g1.reshape(1, H),
                   be1.reshape(1, H), W2)
    a = _agg_kernel(p2, src3, dst3, zero_nd)
    zs, zd = _tc_z(a, dinv, b2.reshape(1, H), lpW1[:H], lpW1[H:],
                   lpb1.reshape(1, H))

    es, ed = _decode_gather_kernel(zs, zd, srcl3, dstl3)
    out = _dec_mlp(es, ed, lpW2, lpb2.reshape(1, HH), lpW3,
                   lpb3.reshape(1, 1))
    return out[:, 0]


# speedup vs baseline: 23.6172x; 1.0348x over previous
import functools

import jax
import jax.numpy as jnp
from jax import lax
from jax.experimental import pallas as pl
from jax.experimental.pallas import tpu as pltpu
from jax.experimental.pallas import tpu_sc as plsc

N = 10000
E = 320000
D = 128
H = 128
HH = H // 2
L = 65536
BN_EPS = 1e-5

NP = 10240
NC = 2
NS = 16
NW = NC * NS
SLAB = NP // NS

EPW = E // NW
CK = 125
NCH = EPW // CK
NBC = 40
NB = NCH // NBC
CKD = 100
NCHD = EPW // CKD

LH = L // 2
LPW = LH // NW
CKL = 128
NCHL = LPW // CKL

_SC_MESH = plsc.VectorSubcoreMesh(
    core_axis_name="c", subcore_axis_name="s", num_cores=NC, num_subcores=NS)


@functools.partial(
    pl.kernel,
    mesh=_SC_MESH,
    out_type=jax.ShapeDtypeStruct((NC, NP), jnp.float32),
    scratch_types=[
        pltpu.VMEM((NCHD, CKD), jnp.int32),
        pltpu.VMEM((CKD,), jnp.float32),
        pltpu.VMEM_SHARED((NP,), jnp.float32),
    ],
)
def _deg_kernel(dst_hbm, zero_hbm, ones_hbm, out_hbm, dst_v, ones_v, acc):
    c = lax.axis_index("c")
    s = lax.axis_index("s")
    tid = c * NS + s
    pltpu.sync_copy(dst_hbm.at[tid], dst_v)
    pltpu.sync_copy(ones_hbm, ones_v)
    slab = pl.ds(s * SLAB, SLAB)
    pltpu.sync_copy(zero_hbm.at[slab], acc.at[slab])
    plsc.subcore_barrier()

    def body(j, carry):
        pltpu.sync_copy(ones_v, acc.at[dst_v.at[j]], add=True)
        return carry

    lax.fori_loop(0, NCHD, body, 0)
    plsc.subcore_barrier()
    pltpu.sync_copy(acc.at[slab], out_hbm.at[c, slab])


@functools.partial(
    pl.kernel,
    mesh=_SC_MESH,
    out_type=jax.ShapeDtypeStruct((NC, NP, D), jnp.float32),
    scratch_types=[
        pltpu.VMEM((NBC, CK), jnp.int32),
        pltpu.VMEM((NBC, CK), jnp.int32),
        pltpu.VMEM((CK, D), jnp.float32),
        pltpu.VMEM((CK, D), jnp.float32),
        pltpu.VMEM_SHARED((NP, D), jnp.float32),
        pltpu.SemaphoreType.DMA,
        pltpu.SemaphoreType.DMA,
    ],
)
def _agg_kernel(p_hbm, src_hbm, dst_hbm, zero_hbm, out_hbm,
                src_v, dst_v, buf0, buf1, acc, sem0, sem1):
    c = lax.axis_index("c")
    s = lax.axis_index("s")
    tid = c * NS + s
    slab = pl.ds(s * SLAB, SLAB)

    @pl.when(c == 0)
    def _():
        pltpu.sync_copy(p_hbm.at[slab], acc.at[slab])

    @pl.when(c == 1)
    def _():
        pltpu.sync_copy(zero_hbm.at[slab], acc.at[slab])

    plsc.subcore_barrier()

    def block(b, carry):
        rows = pl.ds(b * NBC, NBC)
        pltpu.sync_copy(src_hbm.at[tid, rows], src_v)
        pltpu.sync_copy(dst_hbm.at[tid, rows], dst_v)
        pltpu.async_copy(p_hbm.at[src_v.at[0]], buf0, sem0)

        def body(k, carry2):
            j0 = 2 * k
            g1 = pltpu.async_copy(p_hbm.at[src_v.at[j0 + 1]], buf1, sem1)
            pltpu.make_async_copy(p_hbm.at[src_v.at[j0]], buf0, sem0).wait()
            pltpu.sync_copy(buf0, acc.at[dst_v.at[j0]], add=True)
            pltpu.async_copy(p_hbm.at[src_v.at[j0 + 2]], buf0, sem0)
            g1.wait()
            pltpu.sync_copy(buf1, acc.at[dst_v.at[j0 + 1]], add=True)
            return carry2

        lax.fori_loop(0, NBC // 2 - 1, body, 0)
        gl = pltpu.async_copy(p_hbm.at[src_v.at[NBC - 1]], buf1, sem1)
        pltpu.make_async_copy(p_hbm.at[src_v.at[NBC - 2]], buf0, sem0).wait()
        pltpu.sync_copy(buf0, acc.at[dst_v.at[NBC - 2]], add=True)
        gl.wait()
        pltpu.sync_copy(buf1, acc.at[dst_v.at[NBC - 1]], add=True)
        return carry

    lax.fori_loop(0, NB, block, 0)
    plsc.subcore_barrier()
    pltpu.sync_copy(acc.at[slab], out_hbm.at[c, slab])


@functools.partial(
    pl.kernel,
    mesh=_SC_MESH,
    out_type=(jax.ShapeDtypeStruct((LH, D), jnp.float32),
              jax.ShapeDtypeStruct((LH, D), jnp.float32)),
    scratch_types=[
        pltpu.VMEM((NCHL, CKL), jnp.int32),
        pltpu.VMEM((NCHL, CKL), jnp.int32),
        pltpu.VMEM((CKL, D), jnp.float32),
        pltpu.VMEM((CKL, D), jnp.float32),
        pltpu.VMEM((CKL, D), jnp.float32),
        pltpu.VMEM((CKL, D), jnp.float32),
        pltpu.SemaphoreType.DMA,
        pltpu.SemaphoreType.DMA,
        pltpu.SemaphoreType.DMA,
        pltpu.SemaphoreType.DMA,
    ],
)
def _decode_gather_kernel(zs_hbm, zd_hbm, src_hbm, dst_hbm, es_hbm, ed_hbm,
                          src_v, dst_v, bufa0, bufb0, bufa1, bufb1,
                          sa0, sb0, sa1, sb1):
    c = lax.axis_index("c")
    s = lax.axis_index("s")
    tid = c * NS + s
    base = tid * LPW
    pltpu.sync_copy(src_hbm.at[tid], src_v)
    pltpu.sync_copy(dst_hbm.at[tid], dst_v)

    pltpu.async_copy(zs_hbm.at[src_v.at[0]], bufa0, sa0)
    pltpu.async_copy(zd_hbm.at[dst_v.at[0]], bufb0, sb0)

    def emit(j, bufa, bufb, sa, sb):
        rows = pl.ds(base + j * CKL, CKL)
        pltpu.make_async_copy(zs_hbm.at[src_v.at[j]], bufa, sa).wait()
        pltpu.sync_copy(bufa, es_hbm.at[rows])
        pltpu.make_async_copy(zd_hbm.at[dst_v.at[j]], bufb, sb).wait()
        pltpu.sync_copy(bufb, ed_hbm.at[rows])

    def body(k, carry):
        j0 = 2 * k
        pltpu.async_copy(zs_hbm.at[src_v.at[j0 + 1]], bufa1, sa1)
        pltpu.async_copy(zd_hbm.at[dst_v.at[j0 + 1]], bufb1, sb1)
        emit(j0, bufa0, bufb0, sa0, sb0)
        pltpu.async_copy(zs_hbm.at[src_v.at[j0 + 2]], bufa0, sa0)
        pltpu.async_copy(zd_hbm.at[dst_v.at[j0 + 2]], bufb0, sb0)
        emit(j0 + 1, bufa1, bufb1, sa1, sb1)
        return carry

    lax.fori_loop(0, NCHL // 2 - 1, body, 0)
    pltpu.async_copy(zs_hbm.at[src_v.at[NCHL - 1]], bufa1, sa1)
    pltpu.async_copy(zd_hbm.at[dst_v.at[NCHL - 1]], bufb1, sb1)
    emit(NCHL - 2, bufa0, bufb0, sa0, sb0)
    emit(NCHL - 1, bufa1, bufb1, sa1, sb1)


_R = 2048


def _tc_mm0_body(x_ref, w_ref, h_ref):
    h_ref[...] = jnp.dot(x_ref[...], w_ref[...],
                         preferred_element_type=jnp.float32)


_tc_mm0 = pl.pallas_call(
    _tc_mm0_body,
    grid=(NP // _R,),
    in_specs=[
        pl.BlockSpec((_R, D), lambda i: (i, 0)),
        pl.BlockSpec((D, H), lambda i: (0, 0)),
    ],
    out_specs=pl.BlockSpec((_R, H), lambda i: (i, 0)),
    out_shape=jax.ShapeDtypeStruct((NP, H), jnp.float32),
)


def _tc_scale_body(h_ref, degt_ref, p_ref, dinv_ref):
    deg = degt_ref[:, 0:1] + degt_ref[:, 1:2] + 1.0
    di = lax.rsqrt(deg)
    dinv_ref[...] = di
    p_ref[...] = di * h_ref[...]


_tc_scale = pl.pallas_call(
    _tc_scale_body,
    grid=(NP // _R,),
    in_specs=[
        pl.BlockSpec((_R, H), lambda i: (i, 0)),
        pl.BlockSpec((_R, 2), lambda i: (i, 0)),
    ],
    out_specs=[
        pl.BlockSpec((_R, H), lambda i: (i, 0)),
        pl.BlockSpec((_R, 1), lambda i: (i, 0)),
    ],
    out_shape=[
        jax.ShapeDtypeStruct((NP, H), jnp.float32),
        jax.ShapeDtypeStruct((NP, 1), jnp.float32),
    ],
)


def _tc_layer_body(a_ref, dinv_ref, b_ref, g_ref, be_ref, w_ref, p_ref):
    di = dinv_ref[...]
    blk = a_ref[...]
    conv = di * (blk[0] + blk[1]) + b_ref[...]
    bn_scale = g_ref[...] * (1.0 / (1.0 + BN_EPS) ** 0.5)
    r = jnp.maximum(conv * bn_scale + be_ref[...], 0.0)
    p_ref[...] = di * jnp.dot(r, w_ref[...],
                              preferred_element_type=jnp.float32)


_tc_layer = pl.pallas_call(
    _tc_layer_body,
    grid=(NP // _R,),
    in_specs=[
        pl.BlockSpec((NC, _R, D), lambda i: (0, i, 0)),
        pl.BlockSpec((_R, 1), lambda i: (i, 0)),
        pl.BlockSpec((1, H), lambda i: (0, 0)),
        pl.BlockSpec((1, H), lambda i: (0, 0)),
        pl.BlockSpec((1, H), lambda i: (0, 0)),
        pl.BlockSpec((H, H), lambda i: (0, 0)),
    ],
    out_specs=pl.BlockSpec((_R, H), lambda i: (i, 0)),
    out_shape=jax.ShapeDtypeStruct((NP, H), jnp.float32),
)


def _tc_z_body(a_ref, dinv_ref, b_ref, wa_ref, wb_ref, lpb1_ref,
               zs_ref, zd_ref):
    blk = a_ref[...]
    z = dinv_ref[...] * (blk[0] + blk[1]) + b_ref[...]
    zs_ref[...] = jnp.dot(z, wa_ref[...],
                          preferred_element_type=jnp.float32) + lpb1_ref[...]
    zd_ref[...] = jnp.dot(z, wb_ref[...], preferred_element_type=jnp.float32)


_tc_z = pl.pallas_call(
    _tc_z_body,
    grid=(NP // _R,),
    in_specs=[
        pl.BlockSpec((NC, _R, D), lambda i: (0, i, 0)),
        pl.BlockSpec((_R, 1), lambda i: (i, 0)),
        pl.BlockSpec((1, H), lambda i: (0, 0)),
        pl.BlockSpec((H, H), lambda i: (0, 0)),
        pl.BlockSpec((H, H), lambda i: (0, 0)),
        pl.BlockSpec((1, H), lambda i: (0, 0)),
    ],
    out_specs=[
        pl.BlockSpec((_R, H), lambda i: (i, 0)),
        pl.BlockSpec((_R, H), lambda i: (i, 0)),
    ],
    out_shape=[
        jax.ShapeDtypeStruct((NP, H), jnp.float32),
        jax.ShapeDtypeStruct((NP, H), jnp.float32),
    ],
)

_RL = 4096


def _dec_mlp_body(es_ref, ed_ref, w2_ref, b2_ref, w3_ref, b3_ref, out_ref):
    t = jnp.maximum(es_ref[...] + ed_ref[...], 0.0)
    t2 = jnp.maximum(
        jnp.dot(t, w2_ref[...], preferred_element_type=jnp.float32)
        + b2_ref[...], 0.0)
    out_ref[...] = jnp.dot(t2, w3_ref[...],
                           preferred_element_type=jnp.float32) + b3_ref[...]


_dec_mlp = pl.pallas_call(
    _dec_mlp_body,
    grid=(LH // _RL,),
    in_specs=[
        pl.BlockSpec((_RL, D), lambda i: (i, 0)),
        pl.BlockSpec((_RL, D), lambda i: (i, 0)),
        pl.BlockSpec((H, HH), lambda i: (0, 0)),
        pl.BlockSpec((1, HH), lambda i: (0, 0)),
        pl.BlockSpec((HH, 1), lambda i: (0, 0)),
        pl.BlockSpec((1, 1), lambda i: (0, 0)),
    ],
    out_specs=pl.BlockSpec((_RL, 1), lambda i: (i, 0)),
    out_shape=jax.ShapeDtypeStruct((LH, 1), jnp.float32),
)


def kernel(x, edge_index, edge_label_index, W0, b0, W1, b1, W2, b2,
           g0, be0, g1, be1, lpW1, lpb1, lpW2, lpb2, lpW3, lpb3):
    x_p = jnp.zeros((NP, D), jnp.float32).at[:N].set(x)
    src3 = edge_index[0].reshape(NW, NCH, CK)
    dst3 = edge_index[1].reshape(NW, NCH, CK)
    dst3d = edge_index[1].reshape(NW, NCHD, CKD)
    srcl4 = edge_label_index[0].reshape(2, NW, NCHL, CKL)
    dstl4 = edge_label_index[1].reshape(2, NW, NCHL, CKL)

    zero_np = jnp.zeros((NP,), jnp.float32)
    zero_nd = jnp.zeros((NP, D), jnp.float32)
    ones_ck = jnp.ones((CKD,), jnp.float32)

    h0 = _tc_mm0(x_p, W0)
    degp = _deg_kernel(dst3d, zero_np, ones_ck)
    degt = degp.T

    p0, dinv = _tc_scale(h0, degt)
    a = _agg_kernel(p0, src3, dst3, zero_nd)
    p1 = _tc_layer(a, dinv, b0.reshape(1, H), g0.reshape(1, H),
                   be0.reshape(1, H), W1)
    a = _agg_kernel(p1, src3, dst3, zero_nd)
    p2 = _tc_layer(a, dinv, b1.reshape(1, H), g1.reshape(1, H),
                   be1.reshape(1, H), W2)
    a = _agg_kernel(p2, src3, dst3, zero_nd)
    zs, zd = _tc_z(a, dinv, b2.reshape(1, H), lpW1[:H], lpW1[H:],
                   lpb1.reshape(1, H))

    lpb2r = lpb2.reshape(1, HH)
    lpb3r = lpb3.reshape(1, 1)
    es0, ed0 = _decode_gather_kernel(zs, zd, srcl4[0], dstl4[0])
    es1, ed1 = _decode_gather_kernel(zs, zd, srcl4[1], dstl4[1])
    out0 = _dec_mlp(es0, ed0, lpW2, lpb2r, lpW3, lpb3r)
    out1 = _dec_mlp(es1, ed1, lpW2, lpb2r, lpW3, lpb3r)
    return jnp.concatenate([out0[:, 0], out1[:, 0]])


# speedup vs baseline: 23.7756x; 1.0067x over previous
import functools

import jax
import jax.numpy as jnp
from jax import lax
from jax.experimental import pallas as pl
from jax.experimental.pallas import tpu as pltpu
from jax.experimental.pallas import tpu_sc as plsc

N = 10000
E = 320000
D = 128
H = 128
HH = H // 2
L = 65536
BN_EPS = 1e-5

NP = 10240
NC = 2
NS = 16
NW = NC * NS
SLAB = NP // NS

EPW = E // NW
CK = 125
NCH = EPW // CK
NBC = 40
NB = NCH // NBC
CKD = 125
NCHD = EPW // CKD
DGRP = 8

LH = L // 2
LPW = LH // NW
CKL = 128
NCHL = LPW // CKL

_SC_MESH = plsc.VectorSubcoreMesh(
    core_axis_name="c", subcore_axis_name="s", num_cores=NC, num_subcores=NS)


@functools.partial(
    pl.kernel,
    mesh=_SC_MESH,
    out_type=jax.ShapeDtypeStruct((NC, NP), jnp.float32),
    scratch_types=[
        pltpu.VMEM((NCHD, CKD), jnp.int32),
        pltpu.VMEM((CKD,), jnp.float32),
        pltpu.VMEM_SHARED((NP,), jnp.float32),
        pltpu.SemaphoreType.DMA,
    ],
)
def _deg_kernel(dst_hbm, zero_hbm, ones_hbm, out_hbm, dst_v, ones_v, acc, sem):
    c = lax.axis_index("c")
    s = lax.axis_index("s")
    tid = c * NS + s
    pltpu.sync_copy(dst_hbm.at[tid], dst_v)
    pltpu.sync_copy(ones_hbm, ones_v)
    slab = pl.ds(s * SLAB, SLAB)
    pltpu.sync_copy(zero_hbm.at[slab], acc.at[slab])
    plsc.subcore_barrier()

    def body(g, carry):
        j0 = g * DGRP
        for t in range(DGRP):
            pltpu.async_copy(ones_v, acc.at[dst_v.at[j0 + t]], sem, add=True)
        for t in range(DGRP):
            pltpu.make_async_copy(ones_v, acc.at[dst_v.at[j0 + t]],
                                  sem).wait()
        return carry

    lax.fori_loop(0, NCHD // DGRP, body, 0)
    plsc.subcore_barrier()
    pltpu.sync_copy(acc.at[slab], out_hbm.at[c, slab])


@functools.partial(
    pl.kernel,
    mesh=_SC_MESH,
    out_type=jax.ShapeDtypeStruct((NC, NP, D), jnp.float32),
    scratch_types=[
        pltpu.VMEM((NBC, CK), jnp.int32),
        pltpu.VMEM((NBC, CK), jnp.int32),
        pltpu.VMEM((CK, D), jnp.float32),
        pltpu.VMEM((CK, D), jnp.float32),
        pltpu.VMEM_SHARED((NP, D), jnp.float32),
        pltpu.SemaphoreType.DMA,
        pltpu.SemaphoreType.DMA,
    ],
)
def _agg_kernel(p_hbm, src_hbm, dst_hbm, zero_hbm, out_hbm,
                src_v, dst_v, buf0, buf1, acc, sem0, sem1):
    c = lax.axis_index("c")
    s = lax.axis_index("s")
    tid = c * NS + s
    slab = pl.ds(s * SLAB, SLAB)

    @pl.when(c == 0)
    def _():
        pltpu.sync_copy(p_hbm.at[slab], acc.at[slab])

    @pl.when(c == 1)
    def _():
        pltpu.sync_copy(zero_hbm.at[slab], acc.at[slab])

    plsc.subcore_barrier()

    def block(b, carry):
        rows = pl.ds(b * NBC, NBC)
        pltpu.sync_copy(src_hbm.at[tid, rows], src_v)
        pltpu.sync_copy(dst_hbm.at[tid, rows], dst_v)
        pltpu.async_copy(p_hbm.at[src_v.at[0]], buf0, sem0)

        def body(k, carry2):
            j0 = 2 * k
            g1 = pltpu.async_copy(p_hbm.at[src_v.at[j0 + 1]], buf1, sem1)
            pltpu.make_async_copy(p_hbm.at[src_v.at[j0]], buf0, sem0).wait()
            pltpu.sync_copy(buf0, acc.at[dst_v.at[j0]], add=True)
            pltpu.async_copy(p_hbm.at[src_v.at[j0 + 2]], buf0, sem0)
            g1.wait()
            pltpu.sync_copy(buf1, acc.at[dst_v.at[j0 + 1]], add=True)
            return carry2

        lax.fori_loop(0, NBC // 2 - 1, body, 0)
        gl = pltpu.async_copy(p_hbm.at[src_v.at[NBC - 1]], buf1, sem1)
        pltpu.make_async_copy(p_hbm.at[src_v.at[NBC - 2]], buf0, sem0).wait()
        pltpu.sync_copy(buf0, acc.at[dst_v.at[NBC - 2]], add=True)
        gl.wait()
        pltpu.sync_copy(buf1, acc.at[dst_v.at[NBC - 1]], add=True)
        return carry

    lax.fori_loop(0, NB, block, 0)
    plsc.subcore_barrier()
    pltpu.sync_copy(acc.at[slab], out_hbm.at[c, slab])


@functools.partial(
    pl.kernel,
    mesh=_SC_MESH,
    out_type=(jax.ShapeDtypeStruct((LH, D), jnp.float32),
              jax.ShapeDtypeStruct((LH, D), jnp.float32)),
    scratch_types=[
        pltpu.VMEM((NCHL, CKL), jnp.int32),
        pltpu.VMEM((NCHL, CKL), jnp.int32),
        pltpu.VMEM((CKL, D), jnp.float32),
        pltpu.VMEM((CKL, D), jnp.float32),
        pltpu.VMEM((CKL, D), jnp.float32),
        pltpu.VMEM((CKL, D), jnp.float32),
        pltpu.SemaphoreType.DMA,
        pltpu.SemaphoreType.DMA,
        pltpu.SemaphoreType.DMA,
        pltpu.SemaphoreType.DMA,
    ],
)
def _decode_gather_kernel(zs_hbm, zd_hbm, src_hbm, dst_hbm, es_hbm, ed_hbm,
                          src_v, dst_v, bufa0, bufb0, bufa1, bufb1,
                          sa0, sb0, sa1, sb1):
    c = lax.axis_index("c")
    s = lax.axis_index("s")
    tid = c * NS + s
    base = tid * LPW
    pltpu.sync_copy(src_hbm.at[tid], src_v)
    pltpu.sync_copy(dst_hbm.at[tid], dst_v)

    pltpu.async_copy(zs_hbm.at[src_v.at[0]], bufa0, sa0)
    pltpu.async_copy(zd_hbm.at[dst_v.at[0]], bufb0, sb0)

    def emit(j, bufa, bufb, sa, sb):
        rows = pl.ds(base + j * CKL, CKL)
        pltpu.make_async_copy(zs_hbm.at[src_v.at[j]], bufa, sa).wait()
        pltpu.sync_copy(bufa, es_hbm.at[rows])
        pltpu.make_async_copy(zd_hbm.at[dst_v.at[j]], bufb, sb).wait()
        pltpu.sync_copy(bufb, ed_hbm.at[rows])

    def body(k, carry):
        j0 = 2 * k
        pltpu.async_copy(zs_hbm.at[src_v.at[j0 + 1]], bufa1, sa1)
        pltpu.async_copy(zd_hbm.at[dst_v.at[j0 + 1]], bufb1, sb1)
        emit(j0, bufa0, bufb0, sa0, sb0)
        pltpu.async_copy(zs_hbm.at[src_v.at[j0 + 2]], bufa0, sa0)
        pltpu.async_copy(zd_hbm.at[dst_v.at[j0 + 2]], bufb0, sb0)
        emit(j0 + 1, bufa1, bufb1, sa1, sb1)
        return carry

    lax.fori_loop(0, NCHL // 2 - 1, body, 0)
    pltpu.async_copy(zs_hbm.at[src_v.at[NCHL - 1]], bufa1, sa1)
    pltpu.async_copy(zd_hbm.at[dst_v.at[NCHL - 1]], bufb1, sb1)
    emit(NCHL - 2, bufa0, bufb0, sa0, sb0)
    emit(NCHL - 1, bufa1, bufb1, sa1, sb1)


_R = 2048


def _tc_mm0_body(x_ref, w_ref, h_ref):
    h_ref[...] = jnp.dot(x_ref[...], w_ref[...],
                         preferred_element_type=jnp.float32)


_tc_mm0 = pl.pallas_call(
    _tc_mm0_body,
    grid=(NP // _R,),
    in_specs=[
        pl.BlockSpec((_R, D), lambda i: (i, 0)),
        pl.BlockSpec((D, H), lambda i: (0, 0)),
    ],
    out_specs=pl.BlockSpec((_R, H), lambda i: (i, 0)),
    out_shape=jax.ShapeDtypeStruct((NP, H), jnp.float32),
)


def _tc_scale_body(h_ref, degt_ref, p_ref, dinv_ref):
    deg = degt_ref[:, 0:1] + degt_ref[:, 1:2] + 1.0
    di = lax.rsqrt(deg)
    dinv_ref[...] = di
    p_ref[...] = di * h_ref[...]


_tc_scale = pl.pallas_call(
    _tc_scale_body,
    grid=(NP // _R,),
    in_specs=[
        pl.BlockSpec((_R, H), lambda i: (i, 0)),
        pl.BlockSpec((_R, 2), lambda i: (i, 0)),
    ],
    out_specs=[
        pl.BlockSpec((_R, H), lambda i: (i, 0)),
        pl.BlockSpec((_R, 1), lambda i: (i, 0)),
    ],
    out_shape=[
        jax.ShapeDtypeStruct((NP, H), jnp.float32),
        jax.ShapeDtypeStruct((NP, 1), jnp.float32),
    ],
)


def _tc_layer_body(a_ref, dinv_ref, b_ref, g_ref, be_ref, w_ref, p_ref):
    di = dinv_ref[...]
    blk = a_ref[...]
    conv = di * (blk[0] + blk[1]) + b_ref[...]
    bn_scale = g_ref[...] * (1.0 / (1.0 + BN_EPS) ** 0.5)
    r = jnp.maximum(conv * bn_scale + be_ref[...], 0.0)
    p_ref[...] = di * jnp.dot(r, w_ref[...],
                              preferred_element_type=jnp.float32)


_tc_layer = pl.pallas_call(
    _tc_layer_body,
    grid=(NP // _R,),
    in_specs=[
        pl.BlockSpec((NC, _R, D), lambda i: (0, i, 0)),
        pl.BlockSpec((_R, 1), lambda i: (i, 0)),
        pl.BlockSpec((1, H), lambda i: (0, 0)),
        pl.BlockSpec((1, H), lambda i: (0, 0)),
        pl.BlockSpec((1, H), lambda i: (0, 0)),
        pl.BlockSpec((H, H), lambda i: (0, 0)),
    ],
    out_specs=pl.BlockSpec((_R, H), lambda i: (i, 0)),
    out_shape=jax.ShapeDtypeStruct((NP, H), jnp.float32),
)


def _tc_z_body(a_ref, dinv_ref, b_ref, wa_ref, wb_ref, lpb1_ref,
               zs_ref, zd_ref):
    blk = a_ref[...]
    z = dinv_ref[...] * (blk[0] + blk[1]) + b_ref[...]
    zs_ref[...] = jnp.dot(z, wa_ref[...],
                          preferred_element_type=jnp.float32) + lpb1_ref[...]
    zd_ref[...] = jnp.dot(z, wb_ref[...], preferred_element_type=jnp.float32)


_tc_z = pl.pallas_call(
    _tc_z_body,
    grid=(NP // _R,),
    in_specs=[
        pl.BlockSpec((NC, _R, D), lambda i: (0, i, 0)),
        pl.BlockSpec((_R, 1), lambda i: (i, 0)),
        pl.BlockSpec((1, H), lambda i: (0, 0)),
        pl.BlockSpec((H, H), lambda i: (0, 0)),
        pl.BlockSpec((H, H), lambda i: (0, 0)),
        pl.BlockSpec((1, H), lambda i: (0, 0)),
    ],
    out_specs=[
        pl.BlockSpec((_R, H), lambda i: (i, 0)),
        pl.BlockSpec((_R, H), lambda i: (i, 0)),
    ],
    out_shape=[
        jax.ShapeDtypeStruct((NP, H), jnp.float32),
        jax.ShapeDtypeStruct((NP, H), jnp.float32),
    ],
)

_RL = 4096


def _dec_mlp_body(es_ref, ed_ref, w2_ref, b2_ref, w3_ref, b3_ref, out_ref):
    t = jnp.maximum(es_ref[...] + ed_ref[...], 0.0)
    t2 = jnp.maximum(
        jnp.dot(t, w2_ref[...], preferred_element_type=jnp.float32)
        + b2_ref[...], 0.0)
    out_ref[...] = jnp.dot(t2, w3_ref[...],
                           preferred_element_type=jnp.float32) + b3_ref[...]


_dec_mlp = pl.pallas_call(
    _dec_mlp_body,
    grid=(LH // _RL,),
    in_specs=[
        pl.BlockSpec((_RL, D), lambda i: (i, 0)),
        pl.BlockSpec((_RL, D), lambda i: (i, 0)),
        pl.BlockSpec((H, HH), lambda i: (0, 0)),
        pl.BlockSpec((1, HH), lambda i: (0, 0)),
        pl.BlockSpec((HH, 1), lambda i: (0, 0)),
        pl.BlockSpec((1, 1), lambda i: (0, 0)),
    ],
    out_specs=pl.BlockSpec((_RL, 1), lambda i: (i, 0)),
    out_shape=jax.ShapeDtypeStruct((LH, 1), jnp.float32),
)


def kernel(x, edge_index, edge_label_index, W0, b0, W1, b1, W2, b2,
           g0, be0, g1, be1, lpW1, lpb1, lpW2, lpb2, lpW3, lpb3):
    x_p = jnp.zeros((NP, D), jnp.float32).at[:N].set(x)
    src3 = edge_index[0].reshape(NW, NCH, CK)
    dst3 = edge_index[1].reshape(NW, NCH, CK)
    dst3d = edge_index[1].reshape(NW, NCHD, CKD)
    srcl4 = edge_label_index[0].reshape(2, NW, NCHL, CKL)
    dstl4 = edge_label_index[1].reshape(2, NW, NCHL, CKL)

    zero_np = jnp.zeros((NP,), jnp.float32)
    zero_nd = jnp.zeros((NP, D), jnp.float32)
    ones_ck = jnp.ones((CKD,), jnp.float32)

    h0 = _tc_mm0(x_p, W0)
    degp = _deg_kernel(dst3d, zero_np, ones_ck)
    degt = degp.T

    p0, dinv = _tc_scale(h0, degt)
    a = _agg_kernel(p0, src3, dst3, zero_nd)
    p1 = _tc_layer(a, dinv, b0.reshape(1, H), g0.reshape(1, H),
                   be0.reshape(1, H), W1)
    a = _agg_kernel(p1, src3, dst3, zero_nd)
    p2 = _tc_layer(a, dinv, b1.reshape(1, H), g1.reshape(1, H),
                   be1.reshape(1, H), W2)
    a = _agg_kernel(p2, src3, dst3, zero_nd)
    zs, zd = _tc_z(a, dinv, b2.reshape(1, H), lpW1[:H], lpW1[H:],
                   lpb1.reshape(1, H))

    lpb2r = lpb2.reshape(1, HH)
    lpb3r = lpb3.reshape(1, 1)
    es0, ed0 = _decode_gather_kernel(zs, zd, srcl4[0], dstl4[0])
    es1, ed1 = _decode_gather_kernel(zs, zd, srcl4[1], dstl4[1])
    out0 = _dec_mlp(es0, ed0, lpW2, lpb2r, lpW3, lpb3r)
    out1 = _dec_mlp(es1, ed1, lpW2, lpb2r, lpW3, lpb3r)
    return jnp.concatenate([out0[:, 0], out1[:, 0]])


# speedup vs baseline: 24.3251x; 1.0231x over previous
import functools

import jax
import jax.numpy as jnp
from jax import lax
from jax.experimental import pallas as pl
from jax.experimental.pallas import tpu as pltpu
from jax.experimental.pallas import tpu_sc as plsc

N = 10000
E = 320000
D = 128
H = 128
HH = H // 2
L = 65536
BN_EPS = 1e-5

NP = 10240
NC = 2
NS = 16
NW = NC * NS
SLAB = NP // NS

EPW = E // NW
CK = 125
NCH = EPW // CK
NBC = 40
NB = NCH // NBC
CKD = 125
NCHD = EPW // CKD
DGRP = 8

LH = L // 2
LPW = LH // NW
CKL = 128
NCHL = LPW // CKL

_SC_MESH = plsc.VectorSubcoreMesh(
    core_axis_name="c", subcore_axis_name="s", num_cores=NC, num_subcores=NS)


@functools.partial(
    pl.kernel,
    mesh=_SC_MESH,
    out_type=jax.ShapeDtypeStruct((NC, NP), jnp.float32),
    scratch_types=[
        pltpu.VMEM((NCHD, CKD), jnp.int32),
        pltpu.VMEM((CKD,), jnp.float32),
        pltpu.VMEM_SHARED((NP,), jnp.float32),
        pltpu.SemaphoreType.DMA,
    ],
)
def _deg_kernel(dst_hbm, zero_hbm, ones_hbm, out_hbm, dst_v, ones_v, acc, sem):
    c = lax.axis_index("c")
    s = lax.axis_index("s")
    tid = c * NS + s
    pltpu.sync_copy(dst_hbm.at[tid], dst_v)
    pltpu.sync_copy(ones_hbm, ones_v)
    slab = pl.ds(s * SLAB, SLAB)
    pltpu.sync_copy(zero_hbm.at[slab], acc.at[slab])
    plsc.subcore_barrier()

    def body(g, carry):
        j0 = g * DGRP
        for t in range(DGRP):
            pltpu.async_copy(ones_v, acc.at[dst_v.at[j0 + t]], sem, add=True)
        for t in range(DGRP):
            pltpu.make_async_copy(ones_v, acc.at[dst_v.at[j0 + t]],
                                  sem).wait()
        return carry

    lax.fori_loop(0, NCHD // DGRP, body, 0)
    plsc.subcore_barrier()
    pltpu.sync_copy(acc.at[slab], out_hbm.at[c, slab])


@functools.partial(
    pl.kernel,
    mesh=_SC_MESH,
    out_type=jax.ShapeDtypeStruct((NC, NP, D), jnp.float32),
    scratch_types=[
        pltpu.VMEM((NBC, CK), jnp.int32),
        pltpu.VMEM((NBC, CK), jnp.int32),
        pltpu.VMEM((CK, D), jnp.float32),
        pltpu.VMEM((CK, D), jnp.float32),
        pltpu.VMEM_SHARED((NP, D), jnp.float32),
        pltpu.SemaphoreType.DMA,
        pltpu.SemaphoreType.DMA,
    ],
)
def _agg_kernel(p_hbm, src_hbm, dst_hbm, zero_hbm, out_hbm,
                src_v, dst_v, buf0, buf1, acc, sem0, sem1):
    c = lax.axis_index("c")
    s = lax.axis_index("s")
    tid = c * NS + s
    slab = pl.ds(s * SLAB, SLAB)

    @pl.when(c == 0)
    def _():
        pltpu.sync_copy(p_hbm.at[slab], acc.at[slab])

    @pl.when(c == 1)
    def _():
        pltpu.sync_copy(zero_hbm.at[slab], acc.at[slab])

    plsc.subcore_barrier()

    def block(b, carry):
        rows = pl.ds(b * NBC, NBC)
        pltpu.sync_copy(src_hbm.at[tid, rows], src_v)
        pltpu.sync_copy(dst_hbm.at[tid, rows], dst_v)
        pltpu.async_copy(p_hbm.at[src_v.at[0]], buf0, sem0)

        def body(k, carry2):
            j0 = 2 * k
            g1 = pltpu.async_copy(p_hbm.at[src_v.at[j0 + 1]], buf1, sem1)
            pltpu.make_async_copy(p_hbm.at[src_v.at[j0]], buf0, sem0).wait()
            pltpu.sync_copy(buf0, acc.at[dst_v.at[j0]], add=True)
            pltpu.async_copy(p_hbm.at[src_v.at[j0 + 2]], buf0, sem0)
            g1.wait()
            pltpu.sync_copy(buf1, acc.at[dst_v.at[j0 + 1]], add=True)
            return carry2

        lax.fori_loop(0, NBC // 2 - 1, body, 0)
        gl = pltpu.async_copy(p_hbm.at[src_v.at[NBC - 1]], buf1, sem1)
        pltpu.make_async_copy(p_hbm.at[src_v.at[NBC - 2]], buf0, sem0).wait()
        pltpu.sync_copy(buf0, acc.at[dst_v.at[NBC - 2]], add=True)
        gl.wait()
        pltpu.sync_copy(buf1, acc.at[dst_v.at[NBC - 1]], add=True)
        return carry

    lax.fori_loop(0, NB, block, 0)
    plsc.subcore_barrier()
    pltpu.sync_copy(acc.at[slab], out_hbm.at[c, slab])


@functools.partial(
    pl.kernel,
    mesh=_SC_MESH,
    out_type=(jax.ShapeDtypeStruct((LH, D), jnp.float32),
              jax.ShapeDtypeStruct((LH, D), jnp.float32)),
    scratch_types=[
        pltpu.VMEM((NCHL, CKL), jnp.int32),
        pltpu.VMEM((NCHL, CKL), jnp.int32),
        pltpu.VMEM((CKL, D), jnp.float32),
        pltpu.VMEM((CKL, D), jnp.float32),
        pltpu.VMEM((CKL, D), jnp.float32),
        pltpu.VMEM((CKL, D), jnp.float32),
        pltpu.SemaphoreType.DMA,
        pltpu.SemaphoreType.DMA,
        pltpu.SemaphoreType.DMA,
        pltpu.SemaphoreType.DMA,
    ],
)
def _decode_gather_kernel(zs_hbm, zd_hbm, src_hbm, dst_hbm, es_hbm, ed_hbm,
                          src_v, dst_v, bufa0, bufb0, bufa1, bufb1,
                          sa0, sb0, sa1, sb1):
    c = lax.axis_index("c")
    s = lax.axis_index("s")
    tid = c * NS + s
    base = tid * LPW
    pltpu.sync_copy(src_hbm.at[tid], src_v)
    pltpu.sync_copy(dst_hbm.at[tid], dst_v)

    pltpu.async_copy(zs_hbm.at[src_v.at[0]], bufa0, sa0)
    pltpu.async_copy(zd_hbm.at[dst_v.at[0]], bufb0, sb0)

    def emit(j, bufa, bufb, sa, sb):
        rows = pl.ds(base + j * CKL, CKL)
        pltpu.make_async_copy(zs_hbm.at[src_v.at[j]], bufa, sa).wait()
        pltpu.sync_copy(bufa, es_hbm.at[rows])
        pltpu.make_async_copy(zd_hbm.at[dst_v.at[j]], bufb, sb).wait()
        pltpu.sync_copy(bufb, ed_hbm.at[rows])

    def body(k, carry):
        j0 = 2 * k
        pltpu.async_copy(zs_hbm.at[src_v.at[j0 + 1]], bufa1, sa1)
        pltpu.async_copy(zd_hbm.at[dst_v.at[j0 + 1]], bufb1, sb1)
        emit(j0, bufa0, bufb0, sa0, sb0)
        pltpu.async_copy(zs_hbm.at[src_v.at[j0 + 2]], bufa0, sa0)
        pltpu.async_copy(zd_hbm.at[dst_v.at[j0 + 2]], bufb0, sb0)
        emit(j0 + 1, bufa1, bufb1, sa1, sb1)
        return carry

    lax.fori_loop(0, NCHL // 2 - 1, body, 0)
    pltpu.async_copy(zs_hbm.at[src_v.at[NCHL - 1]], bufa1, sa1)
    pltpu.async_copy(zd_hbm.at[dst_v.at[NCHL - 1]], bufb1, sb1)
    emit(NCHL - 2, bufa0, bufb0, sa0, sb0)
    emit(NCHL - 1, bufa1, bufb1, sa1, sb1)


_R = 2048


def _tc_mm0_body(x_ref, w_ref, h_ref):
    h_ref[...] = jnp.dot(x_ref[...], w_ref[...],
                         preferred_element_type=jnp.float32)


_tc_mm0 = pl.pallas_call(
    _tc_mm0_body,
    grid=(NP // _R,),
    in_specs=[
        pl.BlockSpec((_R, D), lambda i: (i, 0)),
        pl.BlockSpec((D, H), lambda i: (0, 0)),
    ],
    out_specs=pl.BlockSpec((_R, H), lambda i: (i, 0)),
    out_shape=jax.ShapeDtypeStruct((NP, H), jnp.float32),
)


def _tc_scale_body(h_ref, degt_ref, p_ref, dinv_ref):
    deg = degt_ref[:, 0:1] + degt_ref[:, 1:2] + 1.0
    di = lax.rsqrt(deg)
    dinv_ref[...] = di
    p_ref[...] = di * h_ref[...]


_tc_scale = pl.pallas_call(
    _tc_scale_body,
    grid=(NP // _R,),
    in_specs=[
        pl.BlockSpec((_R, H), lambda i: (i, 0)),
        pl.BlockSpec((_R, 2), lambda i: (i, 0)),
    ],
    out_specs=[
        pl.BlockSpec((_R, H), lambda i: (i, 0)),
        pl.BlockSpec((_R, 1), lambda i: (i, 0)),
    ],
    out_shape=[
        jax.ShapeDtypeStruct((NP, H), jnp.float32),
        jax.ShapeDtypeStruct((NP, 1), jnp.float32),
    ],
)


def _tc_layer_body(a_ref, dinv_ref, b_ref, g_ref, be_ref, w_ref, p_ref):
    di = dinv_ref[...]
    blk = a_ref[...]
    conv = di * (blk[0] + blk[1]) + b_ref[...]
    bn_scale = g_ref[...] * (1.0 / (1.0 + BN_EPS) ** 0.5)
    r = jnp.maximum(conv * bn_scale + be_ref[...], 0.0)
    p_ref[...] = di * jnp.dot(r, w_ref[...],
                              preferred_element_type=jnp.float32)


_tc_layer = pl.pallas_call(
    _tc_layer_body,
    grid=(NP // _R,),
    in_specs=[
        pl.BlockSpec((NC, _R, D), lambda i: (0, i, 0)),
        pl.BlockSpec((_R, 1), lambda i: (i, 0)),
        pl.BlockSpec((1, H), lambda i: (0, 0)),
        pl.BlockSpec((1, H), lambda i: (0, 0)),
        pl.BlockSpec((1, H), lambda i: (0, 0)),
        pl.BlockSpec((H, H), lambda i: (0, 0)),
    ],
    out_specs=pl.BlockSpec((_R, H), lambda i: (i, 0)),
    out_shape=jax.ShapeDtypeStruct((NP, H), jnp.float32),
)


def _tc_z_body(a_ref, dinv_ref, b_ref, wa_ref, wb_ref, lpb1_ref,
               zs_ref, zd_ref):
    blk = a_ref[...]
    z = dinv_ref[...] * (blk[0] + blk[1]) + b_ref[...]
    zs_ref[...] = jnp.dot(z, wa_ref[...],
                          preferred_element_type=jnp.float32) + lpb1_ref[...]
    zd_ref[...] = jnp.dot(z, wb_ref[...], preferred_element_type=jnp.float32)


_tc_z = pl.pallas_call(
    _tc_z_body,
    grid=(NP // _R,),
    in_specs=[
        pl.BlockSpec((NC, _R, D), lambda i: (0, i, 0)),
        pl.BlockSpec((_R, 1), lambda i: (i, 0)),
        pl.BlockSpec((1, H), lambda i: (0, 0)),
        pl.BlockSpec((H, H), lambda i: (0, 0)),
        pl.BlockSpec((H, H), lambda i: (0, 0)),
        pl.BlockSpec((1, H), lambda i: (0, 0)),
    ],
    out_specs=[
        pl.BlockSpec((_R, H), lambda i: (i, 0)),
        pl.BlockSpec((_R, H), lambda i: (i, 0)),
    ],
    out_shape=[
        jax.ShapeDtypeStruct((NP, H), jnp.float32),
        jax.ShapeDtypeStruct((NP, H), jnp.float32),
    ],
)

_RL = 4096


def _dec_mlp_body(es_ref, ed_ref, w2_ref, b2_ref, w3t_ref, b3_ref, out_ref):
    t = jnp.maximum(es_ref[...] + ed_ref[...], 0.0)
    t2 = jnp.maximum(
        jnp.dot(t, w2_ref[...], preferred_element_type=jnp.float32)
        + b2_ref[...], 0.0)
    out_ref[...] = jnp.sum(t2 * w3t_ref[...], axis=1) + b3_ref[0, 0]


_dec_mlp = pl.pallas_call(
    _dec_mlp_body,
    grid=(LH // _RL,),
    in_specs=[
        pl.BlockSpec((_RL, D), lambda i: (i, 0)),
        pl.BlockSpec((_RL, D), lambda i: (i, 0)),
        pl.BlockSpec((H, HH), lambda i: (0, 0)),
        pl.BlockSpec((1, HH), lambda i: (0, 0)),
        pl.BlockSpec((1, HH), lambda i: (0, 0)),
        pl.BlockSpec((1, 1), lambda i: (0, 0)),
    ],
    out_specs=pl.BlockSpec((_RL,), lambda i: (i,)),
    out_shape=jax.ShapeDtypeStruct((LH,), jnp.float32),
)


def kernel(x, edge_index, edge_label_index, W0, b0, W1, b1, W2, b2,
           g0, be0, g1, be1, lpW1, lpb1, lpW2, lpb2, lpW3, lpb3):
    x_p = jnp.zeros((NP, D), jnp.float32).at[:N].set(x)
    src3 = edge_index[0].reshape(NW, NCH, CK)
    dst3 = edge_index[1].reshape(NW, NCH, CK)
    dst3d = edge_index[1].reshape(NW, NCHD, CKD)
    srcl4 = edge_label_index[0].reshape(2, NW, NCHL, CKL)
    dstl4 = edge_label_index[1].reshape(2, NW, NCHL, CKL)

    zero_np = jnp.zeros((NP,), jnp.float32)
    zero_nd = jnp.zeros((NP, D), jnp.float32)
    ones_ck = jnp.ones((CKD,), jnp.float32)

    h0 = _tc_mm0(x_p, W0)
    degp = _deg_kernel(dst3d, zero_np, ones_ck)
    degt = degp.T

    p0, dinv = _tc_scale(h0, degt)
    a = _agg_kernel(p0, src3, dst3, zero_nd)
    p1 = _tc_layer(a, dinv, b0.reshape(1, H), g0.reshape(1, H),
                   be0.reshape(1, H), W1)
    a = _agg_kernel(p1, src3, dst3, zero_nd)
    p2 = _tc_layer(a, dinv, b1.reshape(1, H), g1.reshape(1, H),
                   be1.reshape(1, H), W2)
    a = _agg_kernel(p2, src3, dst3, zero_nd)
    zs, zd = _tc_z(a, dinv, b2.reshape(1, H), lpW1[:H], lpW1[H:],
                   lpb1.reshape(1, H))

    lpb2r = lpb2.reshape(1, HH)
    lpW3t = lpW3.reshape(1, HH)
    lpb3r = lpb3.reshape(1, 1)
    es0, ed0 = _decode_gather_kernel(zs, zd, srcl4[0], dstl4[0])
    es1, ed1 = _decode_gather_kernel(zs, zd, srcl4[1], dstl4[1])
    out0 = _dec_mlp(es0, ed0, lpW2, lpb2r, lpW3t, lpb3r)
    out1 = _dec_mlp(es1, ed1, lpW2, lpb2r, lpW3t, lpb3r)
    return jnp.concatenate([out0, out1])


# speedup vs baseline: 24.3445x; 1.0008x over previous
import functools

import jax
import jax.numpy as jnp
from jax import lax
from jax.experimental import pallas as pl
from jax.experimental.pallas import tpu as pltpu
from jax.experimental.pallas import tpu_sc as plsc

N = 10000
E = 320000
D = 128
H = 128
HH = H // 2
L = 65536
BN_EPS = 1e-5

NP = 10240
NC = 2
NS = 16
NW = NC * NS
SLAB = NP // NS

EPW = E // NW
CK = 125
NCH = EPW // CK
NBC = 40
NB = NCH // NBC
CKD = 125
NCHD = EPW // CKD
DGRP = 8

LH = L // 2
LPW = LH // NW
CKL = 128
NCHL = LPW // CKL

_SC_MESH = plsc.VectorSubcoreMesh(
    core_axis_name="c", subcore_axis_name="s", num_cores=NC, num_subcores=NS)


@functools.partial(
    pl.kernel,
    mesh=_SC_MESH,
    out_type=jax.ShapeDtypeStruct((NC, NP), jnp.float32),
    scratch_types=[
        pltpu.VMEM((NCHD, CKD), jnp.int32),
        pltpu.VMEM((CKD,), jnp.float32),
        pltpu.VMEM_SHARED((NP,), jnp.float32),
        pltpu.SemaphoreType.DMA,
    ],
)
def _deg_kernel(dst_hbm, zero_hbm, ones_hbm, out_hbm, dst_v, ones_v, acc, sem):
    c = lax.axis_index("c")
    s = lax.axis_index("s")
    tid = c * NS + s
    pltpu.sync_copy(dst_hbm.at[tid], dst_v)
    pltpu.sync_copy(ones_hbm, ones_v)
    slab = pl.ds(s * SLAB, SLAB)
    pltpu.sync_copy(zero_hbm.at[slab], acc.at[slab])
    plsc.subcore_barrier()

    def body(g, carry):
        j0 = g * DGRP
        for t in range(DGRP):
            pltpu.async_copy(ones_v, acc.at[dst_v.at[j0 + t]], sem, add=True)
        for t in range(DGRP):
            pltpu.make_async_copy(ones_v, acc.at[dst_v.at[j0 + t]],
                                  sem).wait()
        return carry

    lax.fori_loop(0, NCHD // DGRP, body, 0)
    plsc.subcore_barrier()
    pltpu.sync_copy(acc.at[slab], out_hbm.at[c, slab])


@functools.partial(
    pl.kernel,
    mesh=_SC_MESH,
    out_type=jax.ShapeDtypeStruct((NC, NP, D), jnp.float32),
    scratch_types=[
        pltpu.VMEM((NBC, CK), jnp.int32),
        pltpu.VMEM((NBC, CK), jnp.int32),
        pltpu.VMEM((CK, D), jnp.float32),
        pltpu.VMEM((CK, D), jnp.float32),
        pltpu.VMEM_SHARED((NP, D), jnp.float32),
        pltpu.SemaphoreType.DMA,
        pltpu.SemaphoreType.DMA,
    ],
)
def _agg_kernel(p_hbm, src_hbm, dst_hbm, zero_hbm, out_hbm,
                src_v, dst_v, buf0, buf1, acc, sem0, sem1):
    c = lax.axis_index("c")
    s = lax.axis_index("s")
    tid = c * NS + s
    slab = pl.ds(s * SLAB, SLAB)

    @pl.when(c == 0)
    def _():
        pltpu.sync_copy(p_hbm.at[slab], acc.at[slab])

    @pl.when(c == 1)
    def _():
        pltpu.sync_copy(zero_hbm.at[slab], acc.at[slab])

    plsc.subcore_barrier()

    def block(b, carry):
        rows = pl.ds(b * NBC, NBC)
        pltpu.sync_copy(src_hbm.at[tid, rows], src_v)
        pltpu.sync_copy(dst_hbm.at[tid, rows], dst_v)
        pltpu.async_copy(p_hbm.at[src_v.at[0]], buf0, sem0)

        def body(k, carry2):
            j0 = 2 * k
            g1 = pltpu.async_copy(p_hbm.at[src_v.at[j0 + 1]], buf1, sem1)
            pltpu.make_async_copy(p_hbm.at[src_v.at[j0]], buf0, sem0).wait()
            pltpu.sync_copy(buf0, acc.at[dst_v.at[j0]], add=True)
            pltpu.async_copy(p_hbm.at[src_v.at[j0 + 2]], buf0, sem0)
            g1.wait()
            pltpu.sync_copy(buf1, acc.at[dst_v.at[j0 + 1]], add=True)
            return carry2

        lax.fori_loop(0, NBC // 2 - 1, body, 0)
        gl = pltpu.async_copy(p_hbm.at[src_v.at[NBC - 1]], buf1, sem1)
        pltpu.make_async_copy(p_hbm.at[src_v.at[NBC - 2]], buf0, sem0).wait()
        pltpu.sync_copy(buf0, acc.at[dst_v.at[NBC - 2]], add=True)
        gl.wait()
        pltpu.sync_copy(buf1, acc.at[dst_v.at[NBC - 1]], add=True)
        return carry

    lax.fori_loop(0, NB, block, 0)
    plsc.subcore_barrier()
    pltpu.sync_copy(acc.at[slab], out_hbm.at[c, slab])


@functools.partial(
    pl.kernel,
    mesh=_SC_MESH,
    out_type=(jax.ShapeDtypeStruct((LH, D), jnp.float32),
              jax.ShapeDtypeStruct((LH, D), jnp.float32)),
    scratch_types=[
        pltpu.VMEM((NCHL, CKL), jnp.int32),
        pltpu.VMEM((NCHL, CKL), jnp.int32),
        pltpu.VMEM((CKL, D), jnp.float32),
        pltpu.VMEM((CKL, D), jnp.float32),
        pltpu.VMEM((CKL, D), jnp.float32),
        pltpu.VMEM((CKL, D), jnp.float32),
        pltpu.SemaphoreType.DMA,
        pltpu.SemaphoreType.DMA,
        pltpu.SemaphoreType.DMA,
        pltpu.SemaphoreType.DMA,
    ],
)
def _decode_gather_kernel(zs_hbm, zd_hbm, src_hbm, dst_hbm, es_hbm, ed_hbm,
                          src_v, dst_v, bufa0, bufb0, bufa1, bufb1,
                          sa0, sb0, sa1, sb1):
    c = lax.axis_index("c")
    s = lax.axis_index("s")
    tid = c * NS + s
    base = tid * LPW
    pltpu.sync_copy(src_hbm.at[tid], src_v)
    pltpu.sync_copy(dst_hbm.at[tid], dst_v)

    pltpu.async_copy(zs_hbm.at[src_v.at[0]], bufa0, sa0)
    pltpu.async_copy(zd_hbm.at[dst_v.at[0]], bufb0, sb0)

    def emit(j, bufa, bufb, sa, sb):
        rows = pl.ds(base + j * CKL, CKL)
        pltpu.make_async_copy(zs_hbm.at[src_v.at[j]], bufa, sa).wait()
        pltpu.sync_copy(bufa, es_hbm.at[rows])
        pltpu.make_async_copy(zd_hbm.at[dst_v.at[j]], bufb, sb).wait()
        pltpu.sync_copy(bufb, ed_hbm.at[rows])

    def body(k, carry):
        j0 = 2 * k
        pltpu.async_copy(zs_hbm.at[src_v.at[j0 + 1]], bufa1, sa1)
        pltpu.async_copy(zd_hbm.at[dst_v.at[j0 + 1]], bufb1, sb1)
        emit(j0, bufa0, bufb0, sa0, sb0)
        pltpu.async_copy(zs_hbm.at[src_v.at[j0 + 2]], bufa0, sa0)
        pltpu.async_copy(zd_hbm.at[dst_v.at[j0 + 2]], bufb0, sb0)
        emit(j0 + 1, bufa1, bufb1, sa1, sb1)
        return carry

    lax.fori_loop(0, NCHL // 2 - 1, body, 0)
    pltpu.async_copy(zs_hbm.at[src_v.at[NCHL - 1]], bufa1, sa1)
    pltpu.async_copy(zd_hbm.at[dst_v.at[NCHL - 1]], bufb1, sb1)
    emit(NCHL - 2, bufa0, bufb0, sa0, sb0)
    emit(NCHL - 1, bufa1, bufb1, sa1, sb1)


_R = 2048


def _tc_mm0_body(x_ref, w_ref, h_ref):
    h_ref[...] = jnp.dot(x_ref[...], w_ref[...],
                         preferred_element_type=jnp.float32)


_tc_mm0 = pl.pallas_call(
    _tc_mm0_body,
    grid=(NP // _R,),
    in_specs=[
        pl.BlockSpec((_R, D), lambda i: (i, 0)),
        pl.BlockSpec((D, H), lambda i: (0, 0)),
    ],
    out_specs=pl.BlockSpec((_R, H), lambda i: (i, 0)),
    out_shape=jax.ShapeDtypeStruct((NP, H), jnp.float32),
)


def _tc_scale_body(h_ref, degt_ref, p_ref):
    deg = degt_ref[:, 0:1] + degt_ref[:, 1:2] + 1.0
    di = lax.rsqrt(deg)
    p_ref[...] = di * h_ref[...]


_tc_scale = pl.pallas_call(
    _tc_scale_body,
    grid=(NP // _R,),
    in_specs=[
        pl.BlockSpec((_R, H), lambda i: (i, 0)),
        pl.BlockSpec((_R, 2), lambda i: (i, 0)),
    ],
    out_specs=pl.BlockSpec((_R, H), lambda i: (i, 0)),
    out_shape=jax.ShapeDtypeStruct((NP, H), jnp.float32),
)


def _tc_layer_body(a_ref, degt_ref, b_ref, g_ref, be_ref, w_ref, p_ref):
    di = lax.rsqrt(degt_ref[:, 0:1] + degt_ref[:, 1:2] + 1.0)
    blk = a_ref[...]
    conv = di * (blk[0] + blk[1]) + b_ref[...]
    bn_scale = g_ref[...] * (1.0 / (1.0 + BN_EPS) ** 0.5)
    r = jnp.maximum(conv * bn_scale + be_ref[...], 0.0)
    p_ref[...] = di * jnp.dot(r, w_ref[...],
                              preferred_element_type=jnp.float32)


_tc_layer = pl.pallas_call(
    _tc_layer_body,
    grid=(NP // _R,),
    in_specs=[
        pl.BlockSpec((NC, _R, D), lambda i: (0, i, 0)),
        pl.BlockSpec((_R, 2), lambda i: (i, 0)),
        pl.BlockSpec((1, H), lambda i: (0, 0)),
        pl.BlockSpec((1, H), lambda i: (0, 0)),
        pl.BlockSpec((1, H), lambda i: (0, 0)),
        pl.BlockSpec((H, H), lambda i: (0, 0)),
    ],
    out_specs=pl.BlockSpec((_R, H), lambda i: (i, 0)),
    out_shape=jax.ShapeDtypeStruct((NP, H), jnp.float32),
)


def _tc_z_body(a_ref, degt_ref, b_ref, wa_ref, wb_ref, lpb1_ref,
               zs_ref, zd_ref):
    di = lax.rsqrt(degt_ref[:, 0:1] + degt_ref[:, 1:2] + 1.0)
    blk = a_ref[...]
    z = di * (blk[0] + blk[1]) + b_ref[...]
    zs_ref[...] = jnp.dot(z, wa_ref[...],
                          preferred_element_type=jnp.float32) + lpb1_ref[...]
    zd_ref[...] = jnp.dot(z, wb_ref[...], preferred_element_type=jnp.float32)


_tc_z = pl.pallas_call(
    _tc_z_body,
    grid=(NP // _R,),
    in_specs=[
        pl.BlockSpec((NC, _R, D), lambda i: (0, i, 0)),
        pl.BlockSpec((_R, 2), lambda i: (i, 0)),
        pl.BlockSpec((1, H), lambda i: (0, 0)),
        pl.BlockSpec((H, H), lambda i: (0, 0)),
        pl.BlockSpec((H, H), lambda i: (0, 0)),
        pl.BlockSpec((1, H), lambda i: (0, 0)),
    ],
    out_specs=[
        pl.BlockSpec((_R, H), lambda i: (i, 0)),
        pl.BlockSpec((_R, H), lambda i: (i, 0)),
    ],
    out_shape=[
        jax.ShapeDtypeStruct((NP, H), jnp.float32),
        jax.ShapeDtypeStruct((NP, H), jnp.float32),
    ],
)

_RL = 4096


def _dec_mlp_body(es_ref, ed_ref, w2_ref, b2_ref, w3t_ref, b3_ref, out_ref):
    t = jnp.maximum(es_ref[...] + ed_ref[...], 0.0)
    t2 = jnp.maximum(
        jnp.dot(t, w2_ref[...], preferred_element_type=jnp.float32)
        + b2_ref[...], 0.0)
    out_ref[...] = jnp.sum(t2 * w3t_ref[...], axis=1) + b3_ref[0, 0]


_dec_mlp = pl.pallas_call(
    _dec_mlp_body,
    grid=(LH // _RL,),
    in_specs=[
        pl.BlockSpec((_RL, D), lambda i: (i, 0)),
        pl.BlockSpec((_RL, D), lambda i: (i, 0)),
        pl.BlockSpec((H, HH), lambda i: (0, 0)),
        pl.BlockSpec((1, HH), lambda i: (0, 0)),
        pl.BlockSpec((1, HH), lambda i: (0, 0)),
        pl.BlockSpec((1, 1), lambda i: (0, 0)),
    ],
    out_specs=pl.BlockSpec((_RL,), lambda i: (i,)),
    out_shape=jax.ShapeDtypeStruct((LH,), jnp.float32),
)


def kernel(x, edge_index, edge_label_index, W0, b0, W1, b1, W2, b2,
           g0, be0, g1, be1, lpW1, lpb1, lpW2, lpb2, lpW3, lpb3):
    x_p = jnp.zeros((NP, D), jnp.float32).at[:N].set(x)
    src3 = edge_index[0].reshape(NW, NCH, CK)
    dst3 = edge_index[1].reshape(NW, NCH, CK)
    dst3d = edge_index[1].reshape(NW, NCHD, CKD)
    srcl4 = edge_label_index[0].reshape(2, NW, NCHL, CKL)
    dstl4 = edge_label_index[1].reshape(2, NW, NCHL, CKL)

    zero_np = jnp.zeros((NP,), jnp.float32)
    zero_nd = jnp.zeros((NP, D), jnp.float32)
    ones_ck = jnp.ones((CKD,), jnp.float32)

    h0 = _tc_mm0(x_p, W0)
    degp = _deg_kernel(dst3d, zero_np, ones_ck)
    degt = degp.T

    p0 = _tc_scale(h0, degt)
    a = _agg_kernel(p0, src3, dst3, zero_nd)
    p1 = _tc_layer(a, degt, b0.reshape(1, H), g0.reshape(1, H),
                   be0.reshape(1, H), W1)
    a = _agg_kernel(p1, src3, dst3, zero_nd)
    p2 = _tc_layer(a, degt, b1.reshape(1, H), g1.reshape(1, H),
                   be1.reshape(1, H), W2)
    a = _agg_kernel(p2, src3, dst3, zero_nd)
    zs, zd = _tc_z(a, degt, b2.reshape(1, H), lpW1[:H], lpW1[H:],
                   lpb1.reshape(1, H))

    lpb2r = lpb2.reshape(1, HH)
    lpW3t = lpW3.reshape(1, HH)
    lpb3r = lpb3.reshape(1, 1)
    es0, ed0 = _decode_gather_kernel(zs, zd, srcl4[0], dstl4[0])
    es1, ed1 = _decode_gather_kernel(zs, zd, srcl4[1], dstl4[1])
    out0 = _dec_mlp(es0, ed0, lpW2, lpb2r, lpW3t, lpb3r)
    out1 = _dec_mlp(es1, ed1, lpW2, lpb2r, lpW3t, lpb3r)
    return jnp.concatenate([out0, out1])


# speedup vs baseline: 24.3711x; 1.0011x over previous
import functools

import jax
import jax.numpy as jnp
from jax import lax
from jax.experimental import pallas as pl
from jax.experimental.pallas import tpu as pltpu
from jax.experimental.pallas import tpu_sc as plsc

N = 10000
E = 320000
D = 128
H = 128
HH = H // 2
L = 65536
BN_EPS = 1e-5

NP = 10240
NC = 2
NS = 16
NW = NC * NS
SLAB = NP // NS

EPW = E // NW
CK = 125
NCH = EPW // CK
NBC = 40
NB = NCH // NBC
CKD = 125
NCHD = EPW // CKD
DGRP = 8

LH = L // 2
LPW = LH // NW
CKL = 128
NCHL = LPW // CKL

_SC_MESH = plsc.VectorSubcoreMesh(
    core_axis_name="c", subcore_axis_name="s", num_cores=NC, num_subcores=NS)


@functools.partial(
    pl.kernel,
    mesh=_SC_MESH,
    out_type=jax.ShapeDtypeStruct((NC, NP), jnp.float32),
    scratch_types=[
        pltpu.VMEM((NCHD, CKD), jnp.int32),
        pltpu.VMEM((CKD,), jnp.float32),
        pltpu.VMEM_SHARED((NP,), jnp.float32),
        pltpu.SemaphoreType.DMA,
    ],
)
def _deg_kernel(dst_hbm, zero_hbm, ones_hbm, out_hbm, dst_v, ones_v, acc, sem):
    c = lax.axis_index("c")
    s = lax.axis_index("s")
    tid = c * NS + s
    pltpu.sync_copy(dst_hbm.at[tid], dst_v)
    pltpu.sync_copy(ones_hbm, ones_v)
    slab = pl.ds(s * SLAB, SLAB)
    pltpu.sync_copy(zero_hbm.at[slab], acc.at[slab])
    plsc.subcore_barrier()

    def body(g, carry):
        j0 = g * DGRP
        for t in range(DGRP):
            pltpu.async_copy(ones_v, acc.at[dst_v.at[j0 + t]], sem, add=True)
        for t in range(DGRP):
            pltpu.make_async_copy(ones_v, acc.at[dst_v.at[j0 + t]],
                                  sem).wait()
        return carry

    lax.fori_loop(0, NCHD // DGRP, body, 0)
    plsc.subcore_barrier()
    pltpu.sync_copy(acc.at[slab], out_hbm.at[c, slab])


@functools.partial(
    pl.kernel,
    mesh=_SC_MESH,
    out_type=jax.ShapeDtypeStruct((NC, NP, D), jnp.float32),
    scratch_types=[
        pltpu.VMEM((NBC, CK), jnp.int32),
        pltpu.VMEM((NBC, CK), jnp.int32),
        pltpu.VMEM((CK, D), jnp.float32),
        pltpu.VMEM((CK, D), jnp.float32),
        pltpu.VMEM_SHARED((NP, D), jnp.float32),
        pltpu.SemaphoreType.DMA,
        pltpu.SemaphoreType.DMA,
    ],
)
def _agg_kernel(p_hbm, src_hbm, dst_hbm, zero_hbm, out_hbm,
                src_v, dst_v, buf0, buf1, acc, sem0, sem1):
    c = lax.axis_index("c")
    s = lax.axis_index("s")
    tid = c * NS + s
    slab = pl.ds(s * SLAB, SLAB)

    @pl.when(c == 0)
    def _():
        pltpu.sync_copy(p_hbm.at[slab], acc.at[slab])

    @pl.when(c == 1)
    def _():
        pltpu.sync_copy(zero_hbm.at[slab], acc.at[slab])

    plsc.subcore_barrier()

    def block(b, carry):
        rows = pl.ds(b * NBC, NBC)
        pltpu.sync_copy(src_hbm.at[tid, rows], src_v)
        pltpu.sync_copy(dst_hbm.at[tid, rows], dst_v)
        pltpu.async_copy(p_hbm.at[src_v.at[0]], buf0, sem0)

        def body(k, carry2):
            j0 = 2 * k
            g1 = pltpu.async_copy(p_hbm.at[src_v.at[j0 + 1]], buf1, sem1)
            pltpu.make_async_copy(p_hbm.at[src_v.at[j0]], buf0, sem0).wait()
            pltpu.sync_copy(buf0, acc.at[dst_v.at[j0]], add=True)
            pltpu.async_copy(p_hbm.at[src_v.at[j0 + 2]], buf0, sem0)
            g1.wait()
            pltpu.sync_copy(buf1, acc.at[dst_v.at[j0 + 1]], add=True)
            return carry2

        lax.fori_loop(0, NBC // 2 - 1, body, 0)
        gl = pltpu.async_copy(p_hbm.at[src_v.at[NBC - 1]], buf1, sem1)
        pltpu.make_async_copy(p_hbm.at[src_v.at[NBC - 2]], buf0, sem0).wait()
        pltpu.sync_copy(buf0, acc.at[dst_v.at[NBC - 2]], add=True)
        gl.wait()
        pltpu.sync_copy(buf1, acc.at[dst_v.at[NBC - 1]], add=True)
        return carry

    lax.fori_loop(0, NB, block, 0)
    plsc.subcore_barrier()
    pltpu.sync_copy(acc.at[slab], out_hbm.at[c, slab])


@functools.partial(
    pl.kernel,
    mesh=_SC_MESH,
    out_type=(jax.ShapeDtypeStruct((LH, D), jnp.float32),
              jax.ShapeDtypeStruct((LH, D), jnp.float32)),
    scratch_types=[
        pltpu.VMEM((NCHL, CKL), jnp.int32),
        pltpu.VMEM((NCHL, CKL), jnp.int32),
        pltpu.VMEM((CKL, D), jnp.float32),
        pltpu.VMEM((CKL, D), jnp.float32),
        pltpu.VMEM((CKL, D), jnp.float32),
        pltpu.VMEM((CKL, D), jnp.float32),
        pltpu.SemaphoreType.DMA,
        pltpu.SemaphoreType.DMA,
        pltpu.SemaphoreType.DMA,
        pltpu.SemaphoreType.DMA,
    ],
)
def _decode_gather_kernel(zs_hbm, zd_hbm, src_hbm, dst_hbm, es_hbm, ed_hbm,
                          src_v, dst_v, bufa0, bufb0, bufa1, bufb1,
                          sa0, sb0, sa1, sb1):
    c = lax.axis_index("c")
    s = lax.axis_index("s")
    tid = c * NS + s
    base = tid * LPW
    pltpu.sync_copy(src_hbm.at[tid], src_v)
    pltpu.sync_copy(dst_hbm.at[tid], dst_v)

    pltpu.async_copy(zs_hbm.at[src_v.at[0]], bufa0, sa0)
    pltpu.async_copy(zd_hbm.at[dst_v.at[0]], bufb0, sb0)

    def emit(j, bufa, bufb, sa, sb):
        rows = pl.ds(base + j * CKL, CKL)
        pltpu.make_async_copy(zs_hbm.at[src_v.at[j]], bufa, sa).wait()
        pltpu.sync_copy(bufa, es_hbm.at[rows])
        pltpu.make_async_copy(zd_hbm.at[dst_v.at[j]], bufb, sb).wait()
        pltpu.sync_copy(bufb, ed_hbm.at[rows])

    def body(k, carry):
        j0 = 2 * k
        pltpu.async_copy(zs_hbm.at[src_v.at[j0 + 1]], bufa1, sa1)
        pltpu.async_copy(zd_hbm.at[dst_v.at[j0 + 1]], bufb1, sb1)
        emit(j0, bufa0, bufb0, sa0, sb0)
        pltpu.async_copy(zs_hbm.at[src_v.at[j0 + 2]], bufa0, sa0)
        pltpu.async_copy(zd_hbm.at[dst_v.at[j0 + 2]], bufb0, sb0)
        emit(j0 + 1, bufa1, bufb1, sa1, sb1)
        return carry

    lax.fori_loop(0, NCHL // 2 - 1, body, 0)
    pltpu.async_copy(zs_hbm.at[src_v.at[NCHL - 1]], bufa1, sa1)
    pltpu.async_copy(zd_hbm.at[dst_v.at[NCHL - 1]], bufb1, sb1)
    emit(NCHL - 2, bufa0, bufb0, sa0, sb0)
    emit(NCHL - 1, bufa1, bufb1, sa1, sb1)


_R = 2048


def _tc_mm0_body(x_ref, w_ref, h_ref):
    h_ref[...] = jnp.dot(x_ref[...], w_ref[...],
                         preferred_element_type=jnp.float32)


_tc_mm0 = pl.pallas_call(
    _tc_mm0_body,
    grid=(NP // _R,),
    in_specs=[
        pl.BlockSpec((_R, D), lambda i: (i, 0)),
        pl.BlockSpec((D, H), lambda i: (0, 0)),
    ],
    out_specs=pl.BlockSpec((_R, H), lambda i: (i, 0)),
    out_shape=jax.ShapeDtypeStruct((NP, H), jnp.float32),
)


def _tc_scale_body(h_ref, degt_ref, p_ref):
    deg = degt_ref[:, 0:1] + degt_ref[:, 1:2] + 1.0
    di = lax.rsqrt(deg)
    p_ref[...] = di * h_ref[...]


_tc_scale = pl.pallas_call(
    _tc_scale_body,
    grid=(NP // _R,),
    in_specs=[
        pl.BlockSpec((_R, H), lambda i: (i, 0)),
        pl.BlockSpec((_R, 2), lambda i: (i, 0)),
    ],
    out_specs=pl.BlockSpec((_R, H), lambda i: (i, 0)),
    out_shape=jax.ShapeDtypeStruct((NP, H), jnp.float32),
)


def _tc_layer_body(a_ref, degt_ref, b_ref, g_ref, be_ref, w_ref, p_ref):
    di = lax.rsqrt(degt_ref[:, 0:1] + degt_ref[:, 1:2] + 1.0)
    blk = a_ref[...]
    conv = di * (blk[0] + blk[1]) + b_ref[...]
    bn_scale = g_ref[...] * (1.0 / (1.0 + BN_EPS) ** 0.5)
    r = jnp.maximum(conv * bn_scale + be_ref[...], 0.0)
    p_ref[...] = di * jnp.dot(r, w_ref[...],
                              preferred_element_type=jnp.float32)


_tc_layer = pl.pallas_call(
    _tc_layer_body,
    grid=(NP // _R,),
    in_specs=[
        pl.BlockSpec((NC, _R, D), lambda i: (0, i, 0)),
        pl.BlockSpec((_R, 2), lambda i: (i, 0)),
        pl.BlockSpec((1, H), lambda i: (0, 0)),
        pl.BlockSpec((1, H), lambda i: (0, 0)),
        pl.BlockSpec((1, H), lambda i: (0, 0)),
        pl.BlockSpec((H, H), lambda i: (0, 0)),
    ],
    out_specs=pl.BlockSpec((_R, H), lambda i: (i, 0)),
    out_shape=jax.ShapeDtypeStruct((NP, H), jnp.float32),
)


def _tc_z_body(a_ref, degt_ref, b_ref, wa_ref, wb_ref, lpb1_ref,
               zs_ref, zd_ref):
    di = lax.rsqrt(degt_ref[:, 0:1] + degt_ref[:, 1:2] + 1.0)
    blk = a_ref[...]
    z = di * (blk[0] + blk[1]) + b_ref[...]
    zs_ref[...] = jnp.dot(z, wa_ref[...],
                          preferred_element_type=jnp.float32) + lpb1_ref[...]
    zd_ref[...] = jnp.dot(z, wb_ref[...], preferred_element_type=jnp.float32)


_tc_z = pl.pallas_call(
    _tc_z_body,
    grid=(NP // _R,),
    in_specs=[
        pl.BlockSpec((NC, _R, D), lambda i: (0, i, 0)),
        pl.BlockSpec((_R, 2), lambda i: (i, 0)),
        pl.BlockSpec((1, H), lambda i: (0, 0)),
        pl.BlockSpec((H, H), lambda i: (0, 0)),
        pl.BlockSpec((H, H), lambda i: (0, 0)),
        pl.BlockSpec((1, H), lambda i: (0, 0)),
    ],
    out_specs=[
        pl.BlockSpec((_R, H), lambda i: (i, 0)),
        pl.BlockSpec((_R, H), lambda i: (i, 0)),
    ],
    out_shape=[
        jax.ShapeDtypeStruct((NP, H), jnp.float32),
        jax.ShapeDtypeStruct((NP, H), jnp.float32),
    ],
)

_RL = 8192


def _dec_mlp_body(es_ref, ed_ref, w2_ref, b2_ref, w3t_ref, b3_ref, out_ref):
    t = jnp.maximum(es_ref[...] + ed_ref[...], 0.0)
    t2 = jnp.maximum(
        jnp.dot(t, w2_ref[...], preferred_element_type=jnp.float32)
        + b2_ref[...], 0.0)
    out_ref[...] = jnp.sum(t2 * w3t_ref[...], axis=1) + b3_ref[0, 0]


_dec_mlp = pl.pallas_call(
    _dec_mlp_body,
    grid=(LH // _RL,),
    in_specs=[
        pl.BlockSpec((_RL, D), lambda i: (i, 0)),
        pl.BlockSpec((_RL, D), lambda i: (i, 0)),
        pl.BlockSpec((H, HH), lambda i: (0, 0)),
        pl.BlockSpec((1, HH), lambda i: (0, 0)),
        pl.BlockSpec((1, HH), lambda i: (0, 0)),
        pl.BlockSpec((1, 1), lambda i: (0, 0)),
    ],
    out_specs=pl.BlockSpec((_RL,), lambda i: (i,)),
    out_shape=jax.ShapeDtypeStruct((LH,), jnp.float32),
)


def kernel(x, edge_index, edge_label_index, W0, b0, W1, b1, W2, b2,
           g0, be0, g1, be1, lpW1, lpb1, lpW2, lpb2, lpW3, lpb3):
    x_p = jnp.zeros((NP, D), jnp.float32).at[:N].set(x)
    src3 = edge_index[0].reshape(NW, NCH, CK)
    dst3 = edge_index[1].reshape(NW, NCH, CK)
    dst3d = edge_index[1].reshape(NW, NCHD, CKD)
    srcl4 = edge_label_index[0].reshape(2, NW, NCHL, CKL)
    dstl4 = edge_label_index[1].reshape(2, NW, NCHL, CKL)

    zero_np = jnp.zeros((NP,), jnp.float32)
    zero_nd = jnp.zeros((NP, D), jnp.float32)
    ones_ck = jnp.ones((CKD,), jnp.float32)

    h0 = _tc_mm0(x_p, W0)
    degp = _deg_kernel(dst3d, zero_np, ones_ck)
    degt = degp.T

    p0 = _tc_scale(h0, degt)
    a = _agg_kernel(p0, src3, dst3, zero_nd)
    p1 = _tc_layer(a, degt, b0.reshape(1, H), g0.reshape(1, H),
                   be0.reshape(1, H), W1)
    a = _agg_kernel(p1, src3, dst3, zero_nd)
    p2 = _tc_layer(a, degt, b1.reshape(1, H), g1.reshape(1, H),
                   be1.reshape(1, H), W2)
    a = _agg_kernel(p2, src3, dst3, zero_nd)
    zs, zd = _tc_z(a, degt, b2.reshape(1, H), lpW1[:H], lpW1[H:],
                   lpb1.reshape(1, H))

    lpb2r = lpb2.reshape(1, HH)
    lpW3t = lpW3.reshape(1, HH)
    lpb3r = lpb3.reshape(1, 1)
    es0, ed0 = _decode_gather_kernel(zs, zd, srcl4[0], dstl4[0])
    es1, ed1 = _decode_gather_kernel(zs, zd, srcl4[1], dstl4[1])
    out0 = _dec_mlp(es0, ed0, lpW2, lpb2r, lpW3t, lpb3r)
    out1 = _dec_mlp(es1, ed1, lpW2, lpb2r, lpW3t, lpb3r)
    return jnp.concatenate([out0, out1])
